# Initial kernel scaffold; baseline (speedup 1.0000x reference)
import jax
import jax.numpy as jnp
from jax.experimental import pallas as pl


def kernel(x, edge_index, edge_attr, Wl1, bl1, Wr1, br1, We1, att1, b1, Wl2, bl2, Wr2, br2, We2, att2, b2, Wh1, bh1, Wh2, bh2):
    raise NotImplementedError("write your pallas kernel here")


# speedup vs baseline: 4.4472x; 4.4472x over previous
import functools
import jax
import jax.numpy as jnp
from jax import lax
from jax.experimental import pallas as pl
from jax.experimental.pallas import tpu as pltpu
from jax.experimental.pallas import tpu_sc as plsc

N = 10000
E = 320000
NP = 10240
NW = 32
RS = 320
RPAD = 328
CAP = 12352
CK = 2000
K1 = 32
K2 = 64

_mesh = plsc.VectorSubcoreMesh(core_axis_name="c", subcore_axis_name="s")
_CP = pltpu.CompilerParams(needs_layout_passes=False)


def _wid():
    return lax.axis_index("s") * 2 + lax.axis_index("c")


def _iota():
    return lax.iota(jnp.int32, 16)


@functools.partial(
    pl.kernel, mesh=_mesh, compiler_params=_CP,
    out_type=[
        jax.ShapeDtypeStruct((NW * CAP,), jnp.int32),
        jax.ShapeDtypeStruct((NW * CAP,), jnp.int32),
        jax.ShapeDtypeStruct((NW * CAP,), jnp.int32),
        jax.ShapeDtypeStruct((NW * 16,), jnp.int32),
        jax.ShapeDtypeStruct((NP,), jnp.float32),
        jax.ShapeDtypeStruct((NP * 16,), jnp.float32),
    ],
    scratch_types=[
        pltpu.VMEM((CK,), jnp.int32),
        pltpu.VMEM((CK,), jnp.int32),
        pltpu.VMEM((CAP,), jnp.int32),
        pltpu.VMEM((CAP,), jnp.int32),
        pltpu.VMEM((CAP,), jnp.int32),
        pltpu.VMEM((32, 128), jnp.float32),
        pltpu.VMEM((32,), jnp.int32),
        pltpu.VMEM((328,), jnp.float32),
        pltpu.VMEM((5248,), jnp.float32),
        pltpu.VMEM((16,), jnp.int32),
        pltpu.SemaphoreType.DMA,
    ],
)
def _sc_route(src_hbm, dst_hbm, ea128_hbm,
              eidL, srcL, dlocL, counts, deg_hbm, sumattr_hbm,
              srcc, dstc, eidS, srcS, dlocS, eatb, rowst, degacc, sumacc,
              cntst, sem):
    w = _wid()
    lo = w * RS

    def pre(g, _):
        base = g * 16 + _iota()
        eidS[pl.ds(g * 16, 16)] = (base * 7919) % E
        srcS[pl.ds(g * 16, 16)] = (base * 9973) % N
        dlocS[pl.ds(g * 16, 16)] = jnp.full((16,), RS, jnp.int32)
        return 0
    lax.fori_loop(0, CAP // 16, pre, 0)

    def zero_deg(i, _):
        degacc[pl.ds(i * 16, 16)] = jnp.zeros((16,), jnp.float32)
        return 0
    lax.fori_loop(0, 328 // 16, zero_deg, 0)

    def zero_sa(i, _):
        sumacc[pl.ds(i * 16, 16)] = jnp.zeros((16,), jnp.float32)
        return 0
    lax.fori_loop(0, 5248 // 16, zero_sa, 0)

    def chunk(k, cnt):
        pltpu.sync_copy(src_hbm.at[pl.ds(k * CK, CK)], srcc)
        pltpu.sync_copy(dst_hbm.at[pl.ds(k * CK, CK)], dstc)

        def grp(g, cnt):
            sv = srcc[pl.ds(g * 16, 16)]
            dv = dstc[pl.ds(g * 16, 16)]
            ev = k * CK + g * 16 + _iota()
            m = jnp.logical_and(dv >= lo, dv < lo + RS)
            cs = plsc.cumsum(m.astype(jnp.int32))
            pos = cnt + cs - 1
            plsc.store_scatter(eidS, [pos], ev, mask=m)
            plsc.store_scatter(srcS, [pos], sv, mask=m)
            plsc.store_scatter(dlocS, [pos], dv - lo, mask=m)
            return cnt + cs[15]
        return lax.fori_loop(0, CK // 16, grp, cnt)

    cnt = lax.fori_loop(0, E // CK, chunk, jnp.int32(0))

    pltpu.sync_copy(eidS, eidL.at[pl.ds(w * CAP, CAP)])
    pltpu.sync_copy(srcS, srcL.at[pl.ds(w * CAP, CAP)])
    pltpu.sync_copy(dlocS, dlocL.at[pl.ds(w * CAP, CAP)])
    cntst[...] = jnp.full((16,), cnt, jnp.int32)
    pltpu.sync_copy(cntst, counts.at[pl.ds(w * 16, 16)])

    trips = (cnt + 31) // 32

    def seg(k, _):
        for g in range(2):
            ev = eidS[pl.ds(k * 32 + g * 16, 16)]
            rowst[pl.ds(g * 16, 16)] = ev >> 3
        pltpu.async_copy(ea128_hbm.at[rowst], eatb, sem).wait()
        for g in range(2):
            ev = eidS[pl.ds(k * 32 + g * 16, 16)]
            dl = dlocS[pl.ds(k * 32 + g * 16, 16)]
            colbase = (ev & 7) * 16
            rowv = g * 16 + _iota()
            plsc.addupdate_scatter(degacc, [dl], jnp.ones((16,), jnp.float32))
            for a in range(16):
                vals = plsc.load_gather(eatb, [rowv, colbase + a])
                plsc.addupdate_scatter(sumacc, [dl * 16 + a], vals)
        return 0
    lax.fori_loop(0, trips, seg, 0)

    pltpu.sync_copy(degacc.at[pl.ds(0, RS)], deg_hbm.at[pl.ds(w * RS, RS)])
    pltpu.sync_copy(sumacc.at[pl.ds(0, RS * 16)],
                    sumattr_hbm.at[pl.ds(w * RS * 16, RS * 16)])


@functools.partial(
    pl.kernel, mesh=_mesh, compiler_params=_CP,
    out_type=[
        jax.ShapeDtypeStruct((NP * 256,), jnp.float32),
        jax.ShapeDtypeStruct((NP * 16,), jnp.float32),
    ],
    scratch_types=[
        pltpu.VMEM((RPAD * 256,), jnp.float32),
        pltpu.VMEM((RPAD * 16,), jnp.float32),
        pltpu.VMEM((K1, 256), jnp.float32),
        pltpu.VMEM((K1, 256), jnp.float32),
        pltpu.VMEM((K1, 256), jnp.float32),
        pltpu.VMEM((K1,), jnp.int32),
        pltpu.VMEM((K1,), jnp.int32),
        pltpu.VMEM((K1,), jnp.int32),
        pltpu.VMEM((K1,), jnp.int32),
        pltpu.VMEM((256,), jnp.float32),
        pltpu.VMEM((16,), jnp.int32),
        pltpu.SemaphoreType.DMA,
        pltpu.SemaphoreType.DMA,
        pltpu.SemaphoreType.DMA,
    ],
)
def _sc_layer1(eidL, srcL, dlocL, counts, xl1_hbm, xr1_hbm, ea1_hbm, att_hbm,
               outsum_hbm, asum_hbm,
               acc, asumacc, xlb, xrb, eab, eidc, srcc, dlocc, dstg, attv,
               cntb, s1, s2, s3):
    w = _wid()
    lo = w * RS
    pltpu.sync_copy(att_hbm, attv)
    pltpu.sync_copy(counts.at[pl.ds(w * 16, 16)], cntb)
    cnt = cntb[pl.ds(0, 16)][0]

    def z1(i, _):
        acc[pl.ds(i * 16, 16)] = jnp.zeros((16,), jnp.float32)
        return 0
    lax.fori_loop(0, RPAD * 256 // 16, z1, 0)

    def z2(i, _):
        asumacc[pl.ds(i * 16, 16)] = jnp.zeros((16,), jnp.float32)
        return 0
    lax.fori_loop(0, RPAD, z2, 0)

    trips = (cnt + K1 - 1) // K1

    def chunk(k, _):
        off = w * CAP + k * K1
        pltpu.sync_copy(eidL.at[pl.ds(off, K1)], eidc)
        pltpu.sync_copy(srcL.at[pl.ds(off, K1)], srcc)
        pltpu.sync_copy(dlocL.at[pl.ds(off, K1)], dlocc)
        for g in range(K1 // 16):
            dstg[pl.ds(g * 16, 16)] = dlocc[pl.ds(g * 16, 16)] + lo
        c1 = pltpu.async_copy(xl1_hbm.at[srcc], xlb, s1)
        c2 = pltpu.async_copy(xr1_hbm.at[dstg], xrb, s2)
        c3 = pltpu.async_copy(ea1_hbm.at[eidc], eab, s3)
        c1.wait(); c2.wait(); c3.wait()

        for g in range(K1 // 16):
            dl = dlocc[pl.ds(g * 16, 16)]
            rowv = g * 16 + _iota()
            for h in range(4):
                def logit_cc(cc, accv):
                    av = attv[pl.ds(h * 64 + cc * 16, 16)]
                    for c in range(16):
                        col = h * 64 + cc * 16 + c
                        xlv = plsc.load_gather(xlb, [rowv, jnp.full((16,), col, jnp.int32)])
                        xrv = plsc.load_gather(xrb, [rowv, jnp.full((16,), col, jnp.int32)])
                        eav = plsc.load_gather(eab, [rowv, jnp.full((16,), col, jnp.int32)])
                        z = xlv + xrv + eav
                        lr = jnp.maximum(z, 0.2 * z)
                        accv = accv + lr * av[c]
                    return accv
                lg = lax.fori_loop(0, 4, logit_cc, jnp.zeros((16,), jnp.float32))
                wv = jnp.exp(lg)
                plsc.addupdate_scatter(asumacc, [dl * 16 + h], wv)

                def accum_cc(cc, _):
                    for c in range(16):
                        col = h * 64 + cc * 16 + c
                        xlv = plsc.load_gather(xlb, [rowv, jnp.full((16,), col, jnp.int32)])
                        plsc.addupdate_scatter(acc, [dl * 256 + col], wv * xlv)
                    return 0
                lax.fori_loop(0, 4, accum_cc, 0)
        return 0
    lax.fori_loop(0, trips, chunk, 0)

    pltpu.sync_copy(acc.at[pl.ds(0, RS * 256)],
                    outsum_hbm.at[pl.ds(w * RS * 256, RS * 256)])
    pltpu.sync_copy(asumacc.at[pl.ds(0, RS * 16)],
                    asum_hbm.at[pl.ds(w * RS * 16, RS * 16)])


@functools.partial(
    pl.kernel, mesh=_mesh, compiler_params=_CP,
    out_type=[
        jax.ShapeDtypeStruct((NP * 64,), jnp.float32),
        jax.ShapeDtypeStruct((NP,), jnp.float32),
    ],
    scratch_types=[
        pltpu.VMEM((RPAD * 64,), jnp.float32),
        pltpu.VMEM((RPAD,), jnp.float32),
        pltpu.VMEM((K2, 128), jnp.float32),
        pltpu.VMEM((K2, 128), jnp.float32),
        pltpu.VMEM((K2, 128), jnp.float32),
        pltpu.VMEM((K2,), jnp.int32),
        pltpu.VMEM((K2,), jnp.int32),
        pltpu.VMEM((K2,), jnp.int32),
        pltpu.VMEM((K2,), jnp.int32),
        pltpu.VMEM((64,), jnp.float32),
        pltpu.VMEM((16,), jnp.int32),
        pltpu.SemaphoreType.DMA,
        pltpu.SemaphoreType.DMA,
        pltpu.SemaphoreType.DMA,
    ],
)
def _sc_layer2(eidL, srcL, dlocL, counts, t2_hbm, ea2_hbm, att_hbm,
               outsum_hbm, asum_hbm,
               acc, asumacc, tsb, tdb, eab, eidc, srcc, dlocc, dstg, attv,
               cntb, s1, s2, s3):
    w = _wid()
    lo = w * RS
    pltpu.sync_copy(att_hbm, attv)
    pltpu.sync_copy(counts.at[pl.ds(w * 16, 16)], cntb)
    cnt = cntb[pl.ds(0, 16)][0]

    def z1(i, _):
        acc[pl.ds(i * 16, 16)] = jnp.zeros((16,), jnp.float32)
        return 0
    lax.fori_loop(0, RPAD * 64 // 16, z1, 0)

    def z2(i, _):
        asumacc[pl.ds(i * 16, 16)] = jnp.zeros((16,), jnp.float32)
        return 0
    lax.fori_loop(0, RPAD // 8 // 2, z2, 0)

    trips = (cnt + K2 - 1) // K2

    def chunk(k, _):
        off = w * CAP + k * K2
        pltpu.sync_copy(eidL.at[pl.ds(off, K2)], eidc)
        pltpu.sync_copy(srcL.at[pl.ds(off, K2)], srcc)
        pltpu.sync_copy(dlocL.at[pl.ds(off, K2)], dlocc)
        for g in range(K2 // 16):
            dstg[pl.ds(g * 16, 16)] = dlocc[pl.ds(g * 16, 16)] + lo
        c1 = pltpu.async_copy(t2_hbm.at[srcc], tsb, s1)
        c2 = pltpu.async_copy(t2_hbm.at[dstg], tdb, s2)
        c3 = pltpu.async_copy(ea2_hbm.at[eidc], eab, s3)
        c1.wait(); c2.wait(); c3.wait()

        for g in range(K2 // 16):
            dl = dlocc[pl.ds(g * 16, 16)]
            rowv = g * 16 + _iota()

            def logit_cc(cc, accv):
                av = attv[pl.ds(cc * 16, 16)]
                for c in range(16):
                    col = cc * 16 + c
                    xlv = plsc.load_gather(tsb, [rowv, jnp.full((16,), col, jnp.int32)])
                    xrv = plsc.load_gather(tdb, [rowv, jnp.full((16,), col + 64, jnp.int32)])
                    eav = plsc.load_gather(eab, [rowv, jnp.full((16,), col, jnp.int32)])
                    z = xlv + xrv + eav
                    lr = jnp.maximum(z, 0.2 * z)
                    accv = accv + lr * av[c]
                return accv
            lg = lax.fori_loop(0, 4, logit_cc, jnp.zeros((16,), jnp.float32))
            wv = jnp.exp(lg)
            plsc.addupdate_scatter(asumacc, [dl], wv)

            def accum_cc(cc, _):
                for c in range(16):
                    col = cc * 16 + c
                    xlv = plsc.load_gather(tsb, [rowv, jnp.full((16,), col, jnp.int32)])
                    plsc.addupdate_scatter(acc, [dl * 64 + col], wv * xlv)
                return 0
            lax.fori_loop(0, 4, accum_cc, 0)
        return 0
    lax.fori_loop(0, trips, chunk, 0)

    pltpu.sync_copy(acc.at[pl.ds(0, RS * 64)],
                    outsum_hbm.at[pl.ds(w * RS * 64, RS * 64)])
    pltpu.sync_copy(asumacc.at[pl.ds(0, RS)], asum_hbm.at[pl.ds(w * RS, RS)])


def _tc_nodes(xp, Wl1, bl1, Wr1, br1):
    def body(x_ref, wl_ref, bl_ref, wr_ref, br_ref, ol_ref, or_ref):
        xb = x_ref[...]
        ol_ref[...] = jnp.dot(xb, wl_ref[...],
                              preferred_element_type=jnp.float32) + bl_ref[...]
        or_ref[...] = jnp.dot(xb, wr_ref[...],
                              preferred_element_type=jnp.float32) + br_ref[...]
    full = lambda s: pl.BlockSpec(s, lambda i: (0, 0))
    return pl.pallas_call(
        body,
        grid=(NP // 1024,),
        in_specs=[pl.BlockSpec((1024, 64), lambda i: (i, 0)),
                  full((64, 256)), full((1, 256)), full((64, 256)), full((1, 256))],
        out_specs=[pl.BlockSpec((1024, 256), lambda i: (i, 0)),
                   pl.BlockSpec((1024, 256), lambda i: (i, 0))],
        out_shape=[jax.ShapeDtypeStruct((NP, 256), jnp.float32),
                   jax.ShapeDtypeStruct((NP, 256), jnp.float32)],
    )(xp, Wl1, bl1.reshape(1, 256), Wr1, br1.reshape(1, 256))


def _tc_edges(eattr, We1, We2p):
    def body(e_ref, w1_ref, w2_ref, o1_ref, o2_ref):
        eb = e_ref[...]
        o1_ref[...] = jnp.dot(eb, w1_ref[...], preferred_element_type=jnp.float32)
        o2_ref[...] = jnp.dot(eb, w2_ref[...], preferred_element_type=jnp.float32)
    full = lambda s: pl.BlockSpec(s, lambda i: (0, 0))
    return pl.pallas_call(
        body,
        grid=(E // 8000,),
        in_specs=[pl.BlockSpec((8000, 16), lambda i: (i, 0)),
                  full((16, 256)), full((16, 128))],
        out_specs=[pl.BlockSpec((8000, 256), lambda i: (i, 0)),
                   pl.BlockSpec((8000, 128), lambda i: (i, 0))],
        out_shape=[jax.ShapeDtypeStruct((E, 256), jnp.float32),
                   jax.ShapeDtypeStruct((E, 128), jnp.float32)],
    )(eattr, We1, We2p)


def _tc_mid(outsum1, asum1, xl1, xr1, sumattr, deg, We1, We2, S, ST, b1,
            att1bc, att2col, Wl2, bl2, Wr2, br2):
    def body(os_ref, as_ref, xl_ref, xr_ref, sa_ref, dg_ref, we1_ref, we2_ref,
             s_ref, st_ref, b1_ref, a1_ref, a2_ref, wl2_ref, bl2_ref, wr2_ref,
             br2_ref, t2_ref, w2_ref):
        f32 = jnp.float32
        ma = sa_ref[...] / jnp.maximum(dg_ref[...], 1.0)
        la1 = jnp.dot(ma, we1_ref[...], preferred_element_type=f32)
        xl = xl_ref[...]
        z = xl + xr_ref[...] + la1
        lr = jnp.maximum(z, 0.2 * z)
        logits = jnp.dot(lr * a1_ref[...], s_ref[...], preferred_element_type=f32)
        wl1 = jnp.exp(logits)
        wl1b = jnp.dot(wl1, st_ref[...], preferred_element_type=f32)
        num = os_ref[...] + wl1b * xl
        den = jnp.dot(as_ref[...][:, :4] + wl1, st_ref[...],
                      preferred_element_type=f32)
        h1 = jnp.maximum(num / den + b1_ref[...], 0.0)
        xl2 = jnp.dot(h1, wl2_ref[...], preferred_element_type=f32) + bl2_ref[...]
        xr2 = jnp.dot(h1, wr2_ref[...], preferred_element_type=f32) + br2_ref[...]
        t2_ref[...] = jnp.concatenate([xl2, xr2], axis=1)
        la2 = jnp.dot(ma, we2_ref[...], preferred_element_type=f32)
        z2 = xl2 + xr2 + la2
        lr2 = jnp.maximum(z2, 0.2 * z2)
        w2_ref[...] = jnp.exp(jnp.dot(lr2, a2_ref[...], preferred_element_type=f32))
    full = lambda s: pl.BlockSpec(s, lambda i: (0, 0))
    blk = lambda s: pl.BlockSpec(s, lambda i: (i, 0))
    return pl.pallas_call(
        body,
        grid=(NP // 1024,),
        in_specs=[blk((1024, 256)), blk((1024, 16)), blk((1024, 256)),
                  blk((1024, 256)), blk((1024, 16)), blk((1024, 1)),
                  full((16, 256)), full((16, 64)), full((256, 4)),
                  full((4, 256)), full((1, 256)), full((1, 256)),
                  full((64, 1)), full((256, 64)), full((1, 64)),
                  full((256, 64)), full((1, 64))],
        out_specs=[blk((1024, 128)), blk((1024, 1))],
        out_shape=[jax.ShapeDtypeStruct((NP, 128), jnp.float32),
                   jax.ShapeDtypeStruct((NP, 1), jnp.float32)],
    )(outsum1, asum1, xl1, xr1, sumattr, deg, We1, We2, S, ST, b1, att1bc,
      att2col, Wl2, bl2, Wr2, br2)


def _tc_head(outsum2, asum2, wl2, T2, b2, Wh1, bh1, Wh2, bh2):
    def body(os_ref, as_ref, w2_ref, t2_ref, b2_ref, wh1_ref, bh1_ref,
             wh2_ref, bh2_ref, y_ref):
        f32 = jnp.float32
        xl2 = t2_ref[...][:, :64]
        w2 = w2_ref[...]
        out2 = (os_ref[...] + w2 * xl2) / (as_ref[...] + w2) + b2_ref[...]
        hh = jnp.maximum(jnp.dot(out2, wh1_ref[...], preferred_element_type=f32)
                         + bh1_ref[...], 0.0)
        y_ref[...] = jnp.dot(hh, wh2_ref[...],
                             preferred_element_type=f32) + bh2_ref[...]
    full = lambda s: pl.BlockSpec(s, lambda i: (0, 0))
    blk = lambda s: pl.BlockSpec(s, lambda i: (i, 0))
    return pl.pallas_call(
        body,
        grid=(NP // 1024,),
        in_specs=[blk((1024, 64)), blk((1024, 1)), blk((1024, 1)),
                  blk((1024, 128)), full((1, 64)), full((64, 64)),
                  full((1, 64)), full((64, 2)), full((1, 2))],
        out_specs=blk((1024, 2)),
        out_shape=jax.ShapeDtypeStruct((NP, 2), jnp.float32),
    )(outsum2, asum2, wl2, T2, b2.reshape(1, 64), Wh1, bh1.reshape(1, 64),
      Wh2, bh2.reshape(1, 2))


def kernel(x, edge_index, edge_attr, Wl1, bl1, Wr1, br1, We1, att1, b1,
           Wl2, bl2, Wr2, br2, We2, att2, b2, Wh1, bh1, Wh2, bh2):
    src = edge_index[0]
    dst = edge_index[1]
    xp = jnp.pad(x, ((0, NP - N), (0, 0)))
    ea128 = edge_attr.reshape(E // 8, 128)
    We2p = jnp.pad(We2, ((0, 0), (0, 64)))

    eidL, srcL, dlocL, counts, deg, sumattr_f = _sc_route(src, dst, ea128)
    sumattr = sumattr_f.reshape(NP, 16)

    xl1, xr1 = _tc_nodes(xp, Wl1, bl1, Wr1, br1)
    ea1, ea2p = _tc_edges(edge_attr, We1, We2p)

    outsum1_f, asum1_f = _sc_layer1(eidL, srcL, dlocL, counts, xl1, xr1, ea1,
                                    att1.reshape(256))
    outsum1 = outsum1_f.reshape(NP, 256)
    asum1 = asum1_f.reshape(NP, 16)

    S = jnp.kron(jnp.eye(4, dtype=jnp.float32), jnp.ones((64, 1), jnp.float32))
    T2, wl2 = _tc_mid(outsum1, asum1, xl1, xr1, sumattr, deg.reshape(NP, 1),
                      We1, We2, S, S.T, b1.reshape(1, 256),
                      att1.reshape(1, 256), att2.reshape(64, 1),
                      Wl2, bl2.reshape(1, 64), Wr2, br2.reshape(1, 64))

    outsum2_f, asum2_f = _sc_layer2(eidL, srcL, dlocL, counts, T2, ea2p,
                                    att2.reshape(64))
    outsum2 = outsum2_f.reshape(NP, 64)
    asum2 = asum2_f.reshape(NP, 1)

    y = _tc_head(outsum2, asum2, wl2, T2, b2, Wh1, bh1, Wh2, bh2)
    return y[:N]


# speedup vs baseline: 4.8221x; 1.0843x over previous
import functools
import jax
import jax.numpy as jnp
from jax import lax
from jax.experimental import pallas as pl
from jax.experimental.pallas import tpu as pltpu
from jax.experimental.pallas import tpu_sc as plsc

N = 10000
E = 320000
NP = 10240
NW = 32
RS = 320
RPAD = 328
CAP = 12800
CK = 2000
SCH = 512

_mesh = plsc.VectorSubcoreMesh(core_axis_name="c", subcore_axis_name="s")
_CP = pltpu.CompilerParams(needs_layout_passes=False)


def _wid():
    return lax.axis_index("s") * 2 + lax.axis_index("c")


def _iota():
    return lax.iota(jnp.int32, 16)


@functools.partial(
    pl.kernel, mesh=_mesh, compiler_params=_CP,
    out_type=[
        jax.ShapeDtypeStruct((NW * CAP,), jnp.int32),
        jax.ShapeDtypeStruct((NW * CAP,), jnp.int32),
        jax.ShapeDtypeStruct((NW * CAP,), jnp.int32),
        jax.ShapeDtypeStruct((NW * 16,), jnp.int32),
        jax.ShapeDtypeStruct((NP,), jnp.float32),
        jax.ShapeDtypeStruct((NP * 16,), jnp.float32),
    ],
    scratch_types=[
        pltpu.VMEM((CK,), jnp.int32),
        pltpu.VMEM((CK,), jnp.int32),
        pltpu.VMEM((CAP,), jnp.int32),
        pltpu.VMEM((CAP,), jnp.int32),
        pltpu.VMEM((CAP,), jnp.int32),
        pltpu.VMEM((64, 128), jnp.float32),
        pltpu.VMEM((328,), jnp.float32),
        pltpu.VMEM((5248,), jnp.float32),
        pltpu.VMEM((16,), jnp.int32),
        pltpu.SemaphoreType.DMA,
    ],
)
def _sc_route(src_hbm, dst_hbm, ea128_hbm,
              eidL, srcL, dlocL, counts, deg_hbm, sumattr_hbm,
              srcc, dstc, eidS, srcS, dlocS, eatb, degacc, sumacc,
              cntst, sem):
    w = _wid()
    lo = w * RS

    def pre(g, _):
        base = g * 16 + _iota()
        eidS[pl.ds(g * 16, 16)] = (base * 7919) % E
        srcS[pl.ds(g * 16, 16)] = (base * 9973) % N
        dlocS[pl.ds(g * 16, 16)] = jnp.full((16,), RS, jnp.int32)
        return 0
    lax.fori_loop(0, CAP // 16, pre, 0)

    def zero_deg(i, _):
        degacc[pl.ds(i * 16, 16)] = jnp.zeros((16,), jnp.float32)
        return 0
    lax.fori_loop(0, 328 // 16, zero_deg, 0)

    def zero_sa(i, _):
        sumacc[pl.ds(i * 16, 16)] = jnp.zeros((16,), jnp.float32)
        return 0
    lax.fori_loop(0, 5248 // 16, zero_sa, 0)

    def chunk(k, cnt):
        pltpu.sync_copy(src_hbm.at[pl.ds(k * CK, CK)], srcc)
        pltpu.sync_copy(dst_hbm.at[pl.ds(k * CK, CK)], dstc)

        def grp(g, cnt):
            sv = srcc[pl.ds(g * 16, 16)]
            dv = dstc[pl.ds(g * 16, 16)]
            ev = k * CK + g * 16 + _iota()
            m = jnp.logical_and(dv >= lo, dv < lo + RS)
            cs = plsc.cumsum(m.astype(jnp.int32))
            pos = cnt + cs - 1
            plsc.store_scatter(eidS, [pos], ev, mask=m)
            plsc.store_scatter(srcS, [pos], sv, mask=m)
            plsc.store_scatter(dlocS, [pos], dv - lo, mask=m)
            return cnt + cs[15]
        return lax.fori_loop(0, CK // 16, grp, cnt)

    cnt = lax.fori_loop(0, E // CK, chunk, jnp.int32(0))

    pltpu.sync_copy(eidS, eidL.at[pl.ds(w * CAP, CAP)])
    pltpu.sync_copy(srcS, srcL.at[pl.ds(w * CAP, CAP)])
    pltpu.sync_copy(dlocS, dlocL.at[pl.ds(w * CAP, CAP)])
    cntst[...] = jnp.full((16,), cnt, jnp.int32)
    pltpu.sync_copy(cntst, counts.at[pl.ds(w * 16, 16)])

    trips = (cnt + 31) // 32

    def seg_issue(k):
        b = (k & 1) * 32
        for g in range(2):
            ev = eidS[pl.ds(k * 32 + g * 16, 16)]
            pltpu.async_copy(ea128_hbm.at[ev >> 3],
                             eatb.at[pl.ds(b + g * 16, 16)], sem)

    def seg(k, _):
        @pl.when(k < trips)
        def _():
            seg_issue(k)

        @pl.when(k > 0)
        def _():
            kp = k - 1
            b = (kp & 1) * 32
            for g in range(2):
                pltpu.make_async_copy(ea128_hbm.at[pl.ds(0, 16)],
                                      eatb.at[pl.ds(b + g * 16, 16)], sem).wait()
            for g in range(2):
                ev = eidS[pl.ds(kp * 32 + g * 16, 16)]
                dl = dlocS[pl.ds(kp * 32 + g * 16, 16)]
                colbase = (ev & 7) * 16
                rowv = b + g * 16 + _iota()
                plsc.addupdate_scatter(degacc, [dl], jnp.ones((16,), jnp.float32))
                for a in range(16):
                    vals = plsc.load_gather(eatb, [rowv, colbase + a])
                    plsc.addupdate_scatter(sumacc, [dl * 16 + a], vals)
        return 0
    lax.fori_loop(0, trips + 1, seg, 0)

    pltpu.sync_copy(degacc.at[pl.ds(0, RS)], deg_hbm.at[pl.ds(w * RS, RS)])
    pltpu.sync_copy(sumacc.at[pl.ds(0, RS * 16)],
                    sumattr_hbm.at[pl.ds(w * RS * 16, RS * 16)])


@functools.partial(
    pl.kernel, mesh=_mesh, compiler_params=_CP,
    out_type=[
        jax.ShapeDtypeStruct((NP * 256,), jnp.float32),
        jax.ShapeDtypeStruct((NP * 16,), jnp.float32),
    ],
    scratch_types=[
        pltpu.VMEM((RPAD * 256,), jnp.float32),
        pltpu.VMEM((RPAD * 16,), jnp.float32),
        pltpu.VMEM((32, 256), jnp.float32),
        pltpu.VMEM((32, 256), jnp.float32),
        pltpu.VMEM((32, 256), jnp.float32),
        pltpu.VMEM((2 * SCH,), jnp.int32),
        pltpu.VMEM((2 * SCH,), jnp.int32),
        pltpu.VMEM((2 * SCH,), jnp.int32),
        pltpu.VMEM((256,), jnp.float32),
        pltpu.VMEM((16,), jnp.int32),
        pltpu.SemaphoreType.DMA,
        pltpu.SemaphoreType.DMA,
        pltpu.SemaphoreType.DMA,
    ],
)
def _sc_layer1(eidL, srcL, dlocL, counts, xl1_hbm, xr1_hbm, ea1_hbm, att_hbm,
               outsum_hbm, asum_hbm,
               acc, asumacc, xlb, xrb, eab, eidS, srcS, dlocS, attv,
               cntb, s1, s2, s3):
    w = _wid()
    lo = w * RS
    pltpu.sync_copy(att_hbm, attv)
    pltpu.sync_copy(counts.at[pl.ds(w * 16, 16)], cntb)
    cnt = cntb[pl.ds(0, 16)][0]

    def z1(i, _):
        acc[pl.ds(i * 16, 16)] = jnp.zeros((16,), jnp.float32)
        return 0
    lax.fori_loop(0, RPAD * 256 // 16, z1, 0)

    def z2(i, _):
        asumacc[pl.ds(i * 16, 16)] = jnp.zeros((16,), jnp.float32)
        return 0
    lax.fori_loop(0, RPAD, z2, 0)

    nchunks = ((cnt + SCH - 1) // SCH) * (SCH // 16)

    def body(k, _):
        @pl.when(k < nchunks)
        def _issue():
            ksup = k >> 5

            @pl.when((k & 31) == 0)
            def _ld():
                soff = w * CAP + ksup * SCH
                sl = (ksup & 1) * SCH
                pltpu.sync_copy(eidL.at[pl.ds(soff, SCH)], eidS.at[pl.ds(sl, SCH)])
                pltpu.sync_copy(srcL.at[pl.ds(soff, SCH)], srcS.at[pl.ds(sl, SCH)])
                pltpu.sync_copy(dlocL.at[pl.ds(soff, SCH)], dlocS.at[pl.ds(sl, SCH)])

            sl = (ksup & 1) * SCH + (k & 31) * 16
            srcv = srcS[pl.ds(sl, 16)]
            dstv = dlocS[pl.ds(sl, 16)] + lo
            eidv = eidS[pl.ds(sl, 16)]
            b = (k & 1) * 16
            pltpu.async_copy(xl1_hbm.at[srcv], xlb.at[pl.ds(b, 16)], s1)
            pltpu.async_copy(xr1_hbm.at[dstv], xrb.at[pl.ds(b, 16)], s2)
            pltpu.async_copy(ea1_hbm.at[eidv], eab.at[pl.ds(b, 16)], s3)

        @pl.when(k > 0)
        def _comp():
            kp = k - 1
            b = (kp & 1) * 16
            pltpu.make_async_copy(xl1_hbm.at[pl.ds(0, 16)],
                                  xlb.at[pl.ds(b, 16)], s1).wait()
            pltpu.make_async_copy(xr1_hbm.at[pl.ds(0, 16)],
                                  xrb.at[pl.ds(b, 16)], s2).wait()
            pltpu.make_async_copy(ea1_hbm.at[pl.ds(0, 16)],
                                  eab.at[pl.ds(b, 16)], s3).wait()
            slp = ((kp >> 5) & 1) * SCH + (kp & 31) * 16
            dlocv = dlocS[pl.ds(slp, 16)]
            rowv = b + _iota()
            dbase = dlocv * 256
            for h in range(4):
                def logit_cc(cc, accv):
                    off = h * 64 + cc * 16
                    av = attv[pl.ds(off, 16)]
                    colb = jnp.full((16,), off, jnp.int32)
                    for c in range(16):
                        colv = colb + c
                        xlv = plsc.load_gather(xlb, [rowv, colv])
                        xrv = plsc.load_gather(xrb, [rowv, colv])
                        eav = plsc.load_gather(eab, [rowv, colv])
                        z = xlv + xrv + eav
                        lr = jnp.maximum(z, 0.2 * z)
                        accv = accv + lr * av[c]
                    return accv
                lg = lax.fori_loop(0, 4, logit_cc, jnp.zeros((16,), jnp.float32))
                wv = jnp.exp(lg)
                plsc.addupdate_scatter(asumacc, [dlocv * 16 + h], wv)

                def accum_cc(cc, _):
                    off = h * 64 + cc * 16
                    colb = jnp.full((16,), off, jnp.int32)
                    for c in range(16):
                        colv = colb + c
                        xlv = plsc.load_gather(xlb, [rowv, colv])
                        plsc.addupdate_scatter(acc, [dbase + colv], wv * xlv)
                    return 0
                lax.fori_loop(0, 4, accum_cc, 0)
        return 0
    lax.fori_loop(0, nchunks + 1, body, 0)

    pltpu.sync_copy(acc.at[pl.ds(0, RS * 256)],
                    outsum_hbm.at[pl.ds(w * RS * 256, RS * 256)])
    pltpu.sync_copy(asumacc.at[pl.ds(0, RS * 16)],
                    asum_hbm.at[pl.ds(w * RS * 16, RS * 16)])


@functools.partial(
    pl.kernel, mesh=_mesh, compiler_params=_CP,
    out_type=[
        jax.ShapeDtypeStruct((NP * 64,), jnp.float32),
        jax.ShapeDtypeStruct((NP,), jnp.float32),
    ],
    scratch_types=[
        pltpu.VMEM((RPAD * 64,), jnp.float32),
        pltpu.VMEM((RPAD,), jnp.float32),
        pltpu.VMEM((RS, 128), jnp.float32),
        pltpu.VMEM((32, 128), jnp.float32),
        pltpu.VMEM((32, 128), jnp.float32),
        pltpu.VMEM((2 * SCH,), jnp.int32),
        pltpu.VMEM((2 * SCH,), jnp.int32),
        pltpu.VMEM((2 * SCH,), jnp.int32),
        pltpu.VMEM((64,), jnp.float32),
        pltpu.VMEM((16,), jnp.int32),
        pltpu.SemaphoreType.DMA,
        pltpu.SemaphoreType.DMA,
    ],
)
def _sc_layer2(eidL, srcL, dlocL, counts, t2_hbm, ea2_hbm, att_hbm,
               outsum_hbm, asum_hbm,
               acc, asumacc, t2r, tsb, eab, eidS, srcS, dlocS, attv,
               cntb, s1, s3):
    w = _wid()
    lo = w * RS
    pltpu.sync_copy(att_hbm, attv)
    pltpu.sync_copy(counts.at[pl.ds(w * 16, 16)], cntb)
    cnt = cntb[pl.ds(0, 16)][0]
    pltpu.sync_copy(t2_hbm.at[pl.ds(lo, RS)], t2r)

    def z1(i, _):
        acc[pl.ds(i * 16, 16)] = jnp.zeros((16,), jnp.float32)
        return 0
    lax.fori_loop(0, RPAD * 64 // 16, z1, 0)

    def z2(i, _):
        asumacc[pl.ds(i * 16, 16)] = jnp.zeros((16,), jnp.float32)
        return 0
    lax.fori_loop(0, RPAD // 8 // 2, z2, 0)

    nchunks = ((cnt + SCH - 1) // SCH) * (SCH // 16)

    def body(k, _):
        @pl.when(k < nchunks)
        def _issue():
            ksup = k >> 5

            @pl.when((k & 31) == 0)
            def _ld():
                soff = w * CAP + ksup * SCH
                sl = (ksup & 1) * SCH
                pltpu.sync_copy(eidL.at[pl.ds(soff, SCH)], eidS.at[pl.ds(sl, SCH)])
                pltpu.sync_copy(srcL.at[pl.ds(soff, SCH)], srcS.at[pl.ds(sl, SCH)])
                pltpu.sync_copy(dlocL.at[pl.ds(soff, SCH)], dlocS.at[pl.ds(sl, SCH)])

            sl = (ksup & 1) * SCH + (k & 31) * 16
            srcv = srcS[pl.ds(sl, 16)]
            eidv = eidS[pl.ds(sl, 16)]
            b = (k & 1) * 16
            pltpu.async_copy(t2_hbm.at[srcv], tsb.at[pl.ds(b, 16)], s1)
            pltpu.async_copy(ea2_hbm.at[eidv], eab.at[pl.ds(b, 16)], s3)

        @pl.when(k > 0)
        def _comp():
            kp = k - 1
            b = (kp & 1) * 16
            pltpu.make_async_copy(t2_hbm.at[pl.ds(0, 16)],
                                  tsb.at[pl.ds(b, 16)], s1).wait()
            pltpu.make_async_copy(ea2_hbm.at[pl.ds(0, 16)],
                                  eab.at[pl.ds(b, 16)], s3).wait()
            slp = ((kp >> 5) & 1) * SCH + (kp & 31) * 16
            dlocv = dlocS[pl.ds(slp, 16)]
            rowv = b + _iota()
            dbase = dlocv * 64

            def logit_cc(cc, accv):
                off = cc * 16
                av = attv[pl.ds(off, 16)]
                colb = jnp.full((16,), off, jnp.int32)
                for c in range(16):
                    colv = colb + c
                    xlv = plsc.load_gather(tsb, [rowv, colv])
                    xrv = plsc.load_gather(t2r, [dlocv, colv + 64])
                    eav = plsc.load_gather(eab, [rowv, colv])
                    z = xlv + xrv + eav
                    lr = jnp.maximum(z, 0.2 * z)
                    accv = accv + lr * av[c]
                return accv
            lg = lax.fori_loop(0, 4, logit_cc, jnp.zeros((16,), jnp.float32))
            wv = jnp.exp(lg)
            plsc.addupdate_scatter(asumacc, [dlocv], wv)

            def accum_cc(cc, _):
                colb = jnp.full((16,), cc * 16, jnp.int32)
                for c in range(16):
                    colv = colb + c
                    xlv = plsc.load_gather(tsb, [rowv, colv])
                    plsc.addupdate_scatter(acc, [dbase + colv], wv * xlv)
                return 0
            lax.fori_loop(0, 4, accum_cc, 0)
        return 0
    lax.fori_loop(0, nchunks + 1, body, 0)

    pltpu.sync_copy(acc.at[pl.ds(0, RS * 64)],
                    outsum_hbm.at[pl.ds(w * RS * 64, RS * 64)])
    pltpu.sync_copy(asumacc.at[pl.ds(0, RS)], asum_hbm.at[pl.ds(w * RS, RS)])


def _tc_nodes(xp, Wl1, bl1, Wr1, br1):
    def body(x_ref, wl_ref, bl_ref, wr_ref, br_ref, ol_ref, or_ref):
        xb = x_ref[...]
        ol_ref[...] = jnp.dot(xb, wl_ref[...],
                              preferred_element_type=jnp.float32, precision=lax.Precision.HIGHEST) + bl_ref[...]
        or_ref[...] = jnp.dot(xb, wr_ref[...],
                              preferred_element_type=jnp.float32, precision=lax.Precision.HIGHEST) + br_ref[...]
    full = lambda s: pl.BlockSpec(s, lambda i: (0, 0))
    return pl.pallas_call(
        body,
        grid=(NP // 1024,),
        in_specs=[pl.BlockSpec((1024, 64), lambda i: (i, 0)),
                  full((64, 256)), full((1, 256)), full((64, 256)), full((1, 256))],
        out_specs=[pl.BlockSpec((1024, 256), lambda i: (i, 0)),
                   pl.BlockSpec((1024, 256), lambda i: (i, 0))],
        out_shape=[jax.ShapeDtypeStruct((NP, 256), jnp.float32),
                   jax.ShapeDtypeStruct((NP, 256), jnp.float32)],
    )(xp, Wl1, bl1.reshape(1, 256), Wr1, br1.reshape(1, 256))


def _tc_edges(eattr, We1, We2p):
    def body(e_ref, w1_ref, w2_ref, o1_ref, o2_ref):
        eb = e_ref[...]
        o1_ref[...] = jnp.dot(eb, w1_ref[...], preferred_element_type=jnp.float32, precision=lax.Precision.HIGHEST)
        o2_ref[...] = jnp.dot(eb, w2_ref[...], preferred_element_type=jnp.float32, precision=lax.Precision.HIGHEST)
    full = lambda s: pl.BlockSpec(s, lambda i: (0, 0))
    return pl.pallas_call(
        body,
        grid=(E // 8000,),
        in_specs=[pl.BlockSpec((8000, 16), lambda i: (i, 0)),
                  full((16, 256)), full((16, 128))],
        out_specs=[pl.BlockSpec((8000, 256), lambda i: (i, 0)),
                   pl.BlockSpec((8000, 128), lambda i: (i, 0))],
        out_shape=[jax.ShapeDtypeStruct((E, 256), jnp.float32),
                   jax.ShapeDtypeStruct((E, 128), jnp.float32)],
    )(eattr, We1, We2p)


def _tc_mid(outsum1, asum1, xl1, xr1, sumattr, deg, We1, We2, S, ST, b1,
            att1bc, att2col, Wl2, bl2, Wr2, br2):
    def body(os_ref, as_ref, xl_ref, xr_ref, sa_ref, dg_ref, we1_ref, we2_ref,
             s_ref, st_ref, b1_ref, a1_ref, a2_ref, wl2_ref, bl2_ref, wr2_ref,
             br2_ref, t2_ref, w2_ref):
        f32 = jnp.float32
        ma = sa_ref[...] / jnp.maximum(dg_ref[...], 1.0)
        la1 = jnp.dot(ma, we1_ref[...], preferred_element_type=f32, precision=lax.Precision.HIGHEST)
        xl = xl_ref[...]
        z = xl + xr_ref[...] + la1
        lr = jnp.maximum(z, 0.2 * z)
        logits = jnp.dot(lr * a1_ref[...], s_ref[...], preferred_element_type=f32, precision=lax.Precision.HIGHEST)
        wl1 = jnp.exp(logits)
        wl1b = jnp.dot(wl1, st_ref[...], preferred_element_type=f32, precision=lax.Precision.HIGHEST)
        num = os_ref[...] + wl1b * xl
        den = jnp.dot(as_ref[...][:, :4] + wl1, st_ref[...],
                      preferred_element_type=f32, precision=lax.Precision.HIGHEST)
        h1 = jnp.maximum(num / den + b1_ref[...], 0.0)
        xl2 = jnp.dot(h1, wl2_ref[...], preferred_element_type=f32, precision=lax.Precision.HIGHEST) + bl2_ref[...]
        xr2 = jnp.dot(h1, wr2_ref[...], preferred_element_type=f32, precision=lax.Precision.HIGHEST) + br2_ref[...]
        t2_ref[...] = jnp.concatenate([xl2, xr2], axis=1)
        la2 = jnp.dot(ma, we2_ref[...], preferred_element_type=f32, precision=lax.Precision.HIGHEST)
        z2 = xl2 + xr2 + la2
        lr2 = jnp.maximum(z2, 0.2 * z2)
        w2_ref[...] = jnp.exp(jnp.dot(lr2, a2_ref[...], preferred_element_type=f32, precision=lax.Precision.HIGHEST))
    full = lambda s: pl.BlockSpec(s, lambda i: (0, 0))
    blk = lambda s: pl.BlockSpec(s, lambda i: (i, 0))
    return pl.pallas_call(
        body,
        grid=(NP // 1024,),
        in_specs=[blk((1024, 256)), blk((1024, 16)), blk((1024, 256)),
                  blk((1024, 256)), blk((1024, 16)), blk((1024, 1)),
                  full((16, 256)), full((16, 64)), full((256, 4)),
                  full((4, 256)), full((1, 256)), full((1, 256)),
                  full((64, 1)), full((256, 64)), full((1, 64)),
                  full((256, 64)), full((1, 64))],
        out_specs=[blk((1024, 128)), blk((1024, 1))],
        out_shape=[jax.ShapeDtypeStruct((NP, 128), jnp.float32),
                   jax.ShapeDtypeStruct((NP, 1), jnp.float32)],
    )(outsum1, asum1, xl1, xr1, sumattr, deg, We1, We2, S, ST, b1, att1bc,
      att2col, Wl2, bl2, Wr2, br2)


def _tc_head(outsum2, asum2, wl2, T2, b2, Wh1, bh1, Wh2, bh2):
    def body(os_ref, as_ref, w2_ref, t2_ref, b2_ref, wh1_ref, bh1_ref,
             wh2_ref, bh2_ref, y_ref):
        f32 = jnp.float32
        xl2 = t2_ref[...][:, :64]
        w2 = w2_ref[...]
        out2 = (os_ref[...] + w2 * xl2) / (as_ref[...] + w2) + b2_ref[...]
        hh = jnp.maximum(jnp.dot(out2, wh1_ref[...], preferred_element_type=f32, precision=lax.Precision.HIGHEST)
                         + bh1_ref[...], 0.0)
        y_ref[...] = jnp.dot(hh, wh2_ref[...],
                             preferred_element_type=f32, precision=lax.Precision.HIGHEST) + bh2_ref[...]
    full = lambda s: pl.BlockSpec(s, lambda i: (0, 0))
    blk = lambda s: pl.BlockSpec(s, lambda i: (i, 0))
    return pl.pallas_call(
        body,
        grid=(NP // 1024,),
        in_specs=[blk((1024, 64)), blk((1024, 1)), blk((1024, 1)),
                  blk((1024, 128)), full((1, 64)), full((64, 64)),
                  full((1, 64)), full((64, 2)), full((1, 2))],
        out_specs=blk((1024, 2)),
        out_shape=jax.ShapeDtypeStruct((NP, 2), jnp.float32),
    )(outsum2, asum2, wl2, T2, b2.reshape(1, 64), Wh1, bh1.reshape(1, 64),
      Wh2, bh2.reshape(1, 2))


def kernel(x, edge_index, edge_attr, Wl1, bl1, Wr1, br1, We1, att1, b1,
           Wl2, bl2, Wr2, br2, We2, att2, b2, Wh1, bh1, Wh2, bh2):
    src = edge_index[0]
    dst = edge_index[1]
    xp = jnp.pad(x, ((0, NP - N), (0, 0)))
    ea128 = edge_attr.reshape(E // 8, 128)
    We2p = jnp.pad(We2, ((0, 0), (0, 64)))

    eidL, srcL, dlocL, counts, deg, sumattr_f = _sc_route(src, dst, ea128)
    sumattr = sumattr_f.reshape(NP, 16)

    xl1, xr1 = _tc_nodes(xp, Wl1, bl1, Wr1, br1)
    ea1, ea2p = _tc_edges(edge_attr, We1, We2p)

    outsum1_f, asum1_f = _sc_layer1(eidL, srcL, dlocL, counts, xl1, xr1, ea1,
                                    att1.reshape(256))
    outsum1 = outsum1_f.reshape(NP, 256)
    asum1 = asum1_f.reshape(NP, 16)

    S = jnp.kron(jnp.eye(4, dtype=jnp.float32), jnp.ones((64, 1), jnp.float32))
    T2, wl2 = _tc_mid(outsum1, asum1, xl1, xr1, sumattr, deg.reshape(NP, 1),
                      We1, We2, S, S.T, b1.reshape(1, 256),
                      att1.reshape(1, 256), att2.reshape(64, 1),
                      Wl2, bl2.reshape(1, 64), Wr2, br2.reshape(1, 64))

    outsum2_f, asum2_f = _sc_layer2(eidL, srcL, dlocL, counts, T2, ea2p,
                                    att2.reshape(64))
    outsum2 = outsum2_f.reshape(NP, 64)
    asum2 = asum2_f.reshape(NP, 1)

    y = _tc_head(outsum2, asum2, wl2, T2, b2, Wh1, bh1, Wh2, bh2)
    return y[:N]


# speedup vs baseline: 11.1549x; 2.3133x over previous
import functools
import jax
import jax.numpy as jnp
from jax import lax
from jax.experimental import pallas as pl
from jax.experimental.pallas import tpu as pltpu
from jax.experimental.pallas import tpu_sc as plsc

N = 10000
E = 320000
NP = 10240
NW = 32
RS = 320
RPAD = 328
CAP = 12800
CK = 2000
SCH = 512

_mesh = plsc.VectorSubcoreMesh(core_axis_name="c", subcore_axis_name="s")
_CP = pltpu.CompilerParams(needs_layout_passes=False)


def _wid():
    return lax.axis_index("s") * 2 + lax.axis_index("c")


def _iota():
    return lax.iota(jnp.int32, 16)


@functools.partial(
    pl.kernel, mesh=_mesh, compiler_params=_CP,
    out_type=[
        jax.ShapeDtypeStruct((NW * CAP,), jnp.int32),
        jax.ShapeDtypeStruct((NW * CAP,), jnp.int32),
        jax.ShapeDtypeStruct((NW * CAP,), jnp.int32),
        jax.ShapeDtypeStruct((NW * 16,), jnp.int32),
        jax.ShapeDtypeStruct((NP,), jnp.float32),
        jax.ShapeDtypeStruct((NP * 16,), jnp.float32),
    ],
    scratch_types=[
        pltpu.VMEM((CK,), jnp.int32),
        pltpu.VMEM((CK,), jnp.int32),
        pltpu.VMEM((CAP,), jnp.int32),
        pltpu.VMEM((CAP,), jnp.int32),
        pltpu.VMEM((CAP,), jnp.int32),
        pltpu.VMEM((64, 128), jnp.float32),
        pltpu.VMEM((328,), jnp.float32),
        pltpu.VMEM((5248,), jnp.float32),
        pltpu.VMEM((16,), jnp.int32),
        pltpu.SemaphoreType.DMA,
    ],
)
def _sc_route(src_hbm, dst_hbm, ea128_hbm,
              eidL, srcL, dlocL, counts, deg_hbm, sumattr_hbm,
              srcc, dstc, eidS, srcS, dlocS, eatb, degacc, sumacc,
              cntst, sem):
    w = _wid()
    lo = w * RS

    def pre(g, _):
        base = g * 16 + _iota()
        eidS[pl.ds(g * 16, 16)] = (base * 7919) % E
        srcS[pl.ds(g * 16, 16)] = (base * 9973) % N
        dlocS[pl.ds(g * 16, 16)] = jnp.full((16,), RS, jnp.int32)
        return 0
    lax.fori_loop(0, CAP // 16, pre, 0)

    def zero_deg(i, _):
        degacc[pl.ds(i * 16, 16)] = jnp.zeros((16,), jnp.float32)
        return 0
    lax.fori_loop(0, 328 // 16, zero_deg, 0)

    def zero_sa(i, _):
        sumacc[pl.ds(i * 16, 16)] = jnp.zeros((16,), jnp.float32)
        return 0
    lax.fori_loop(0, 5248 // 16, zero_sa, 0)

    def chunk(k, cnt):
        pltpu.sync_copy(src_hbm.at[pl.ds(k * CK, CK)], srcc)
        pltpu.sync_copy(dst_hbm.at[pl.ds(k * CK, CK)], dstc)

        def grp(g, cnt):
            sv = srcc[pl.ds(g * 16, 16)]
            dv = dstc[pl.ds(g * 16, 16)]
            ev = k * CK + g * 16 + _iota()
            m = jnp.logical_and(dv >= lo, dv < lo + RS)
            cs = plsc.cumsum(m.astype(jnp.int32))
            pos = cnt + cs - 1
            plsc.store_scatter(eidS, [pos], ev, mask=m)
            plsc.store_scatter(srcS, [pos], sv, mask=m)
            plsc.store_scatter(dlocS, [pos], dv - lo, mask=m)
            return cnt + cs[15]
        return lax.fori_loop(0, CK // 16, grp, cnt)

    cnt = lax.fori_loop(0, E // CK, chunk, jnp.int32(0))

    pltpu.sync_copy(eidS, eidL.at[pl.ds(w * CAP, CAP)])
    pltpu.sync_copy(srcS, srcL.at[pl.ds(w * CAP, CAP)])
    pltpu.sync_copy(dlocS, dlocL.at[pl.ds(w * CAP, CAP)])
    cntst[...] = jnp.full((16,), cnt, jnp.int32)
    pltpu.sync_copy(cntst, counts.at[pl.ds(w * 16, 16)])

    trips = (cnt + 31) // 32

    def seg_issue(k):
        b = (k & 1) * 32
        for g in range(2):
            ev = eidS[pl.ds(k * 32 + g * 16, 16)]
            pltpu.async_copy(ea128_hbm.at[ev >> 3],
                             eatb.at[pl.ds(b + g * 16, 16)], sem)

    def seg(k, _):
        @pl.when(k < trips)
        def _():
            seg_issue(k)

        @pl.when(k > 0)
        def _():
            kp = k - 1
            b = (kp & 1) * 32
            for g in range(2):
                pltpu.make_async_copy(ea128_hbm.at[pl.ds(0, 16)],
                                      eatb.at[pl.ds(b + g * 16, 16)], sem).wait()
            for g in range(2):
                ev = eidS[pl.ds(kp * 32 + g * 16, 16)]
                dl = dlocS[pl.ds(kp * 32 + g * 16, 16)]
                colbase = (ev & 7) * 16
                plsc.addupdate_scatter(degacc, [dl], jnp.ones((16,), jnp.float32))
                for e in range(16):
                    vals = eatb[b + g * 16 + e, pl.ds(colbase[e], 16)]
                    plsc.addupdate(sumacc.at[pl.ds(dl[e] * 16, 16)], vals)
        return 0
    lax.fori_loop(0, trips + 1, seg, 0)

    pltpu.sync_copy(degacc.at[pl.ds(0, RS)], deg_hbm.at[pl.ds(w * RS, RS)])
    pltpu.sync_copy(sumacc.at[pl.ds(0, RS * 16)],
                    sumattr_hbm.at[pl.ds(w * RS * 16, RS * 16)])


@functools.partial(
    pl.kernel, mesh=_mesh, compiler_params=_CP,
    out_type=[
        jax.ShapeDtypeStruct((NP * 256,), jnp.float32),
        jax.ShapeDtypeStruct((NP * 16,), jnp.float32),
    ],
    scratch_types=[
        pltpu.VMEM((RPAD * 256,), jnp.float32),
        pltpu.VMEM((RPAD * 16,), jnp.float32),
        pltpu.VMEM((32, 256), jnp.float32),
        pltpu.VMEM((32, 256), jnp.float32),
        pltpu.VMEM((32, 256), jnp.float32),
        pltpu.VMEM((2 * SCH,), jnp.int32),
        pltpu.VMEM((2 * SCH,), jnp.int32),
        pltpu.VMEM((2 * SCH,), jnp.int32),
        pltpu.VMEM((256,), jnp.float32),
        pltpu.VMEM((16,), jnp.int32),
        pltpu.SemaphoreType.DMA,
        pltpu.SemaphoreType.DMA,
        pltpu.SemaphoreType.DMA,
    ],
)
def _sc_layer1(eidL, srcL, dlocL, counts, xl1_hbm, xr1_hbm, ea1_hbm, att_hbm,
               outsum_hbm, asum_hbm,
               acc, asumacc, xlb, xrb, eab, eidS, srcS, dlocS, attv,
               cntb, s1, s2, s3):
    w = _wid()
    lo = w * RS
    pltpu.sync_copy(att_hbm, attv)
    pltpu.sync_copy(counts.at[pl.ds(w * 16, 16)], cntb)
    cnt = cntb[pl.ds(0, 16)][0]

    def z1(i, _):
        acc[pl.ds(i * 16, 16)] = jnp.zeros((16,), jnp.float32)
        return 0
    lax.fori_loop(0, RPAD * 256 // 16, z1, 0)

    def z2(i, _):
        asumacc[pl.ds(i * 16, 16)] = jnp.zeros((16,), jnp.float32)
        return 0
    lax.fori_loop(0, RPAD, z2, 0)

    nchunks = ((cnt + SCH - 1) // SCH) * (SCH // 16)

    def body(k, _):
        @pl.when(k < nchunks)
        def _issue():
            ksup = k >> 5

            @pl.when((k & 31) == 0)
            def _ld():
                soff = w * CAP + ksup * SCH
                sl = (ksup & 1) * SCH
                pltpu.sync_copy(eidL.at[pl.ds(soff, SCH)], eidS.at[pl.ds(sl, SCH)])
                pltpu.sync_copy(srcL.at[pl.ds(soff, SCH)], srcS.at[pl.ds(sl, SCH)])
                pltpu.sync_copy(dlocL.at[pl.ds(soff, SCH)], dlocS.at[pl.ds(sl, SCH)])

            sl = (ksup & 1) * SCH + (k & 31) * 16
            srcv = srcS[pl.ds(sl, 16)]
            dstv = dlocS[pl.ds(sl, 16)] + lo
            eidv = eidS[pl.ds(sl, 16)]
            b = (k & 1) * 16
            pltpu.async_copy(xl1_hbm.at[srcv], xlb.at[pl.ds(b, 16)], s1)
            pltpu.async_copy(xr1_hbm.at[dstv], xrb.at[pl.ds(b, 16)], s2)
            pltpu.async_copy(ea1_hbm.at[eidv], eab.at[pl.ds(b, 16)], s3)

        @pl.when(k > 0)
        def _comp():
            kp = k - 1
            b = (kp & 1) * 16
            pltpu.make_async_copy(xl1_hbm.at[pl.ds(0, 16)],
                                  xlb.at[pl.ds(b, 16)], s1).wait()
            pltpu.make_async_copy(xr1_hbm.at[pl.ds(0, 16)],
                                  xrb.at[pl.ds(b, 16)], s2).wait()
            pltpu.make_async_copy(ea1_hbm.at[pl.ds(0, 16)],
                                  eab.at[pl.ds(b, 16)], s3).wait()
            slp = ((kp >> 5) & 1) * SCH + (kp & 31) * 16
            dlocv = dlocS[pl.ds(slp, 16)]
            lanes = _iota()
            for e in range(16):
                row = b + e
                dloc = dlocv[e]
                lg4 = jnp.zeros((16,), jnp.float32)
                for h in range(4):
                    hacc = jnp.zeros((16,), jnp.float32)
                    for q in range(4):
                        off = h * 64 + q * 16
                        z = (xlb[row, pl.ds(off, 16)] + xrb[row, pl.ds(off, 16)]
                             + eab[row, pl.ds(off, 16)])
                        lr = jnp.maximum(z, 0.2 * z)
                        hacc = hacc + lr * attv[pl.ds(off, 16)]
                    s = lax.reduce_sum(hacc, axes=(0,))
                    lg4 = lg4 + jnp.where(lanes == h, jnp.full((16,), s), 0.0)
                wvec = jnp.exp(lg4)
                w4m = jnp.where(lanes < 4, wvec, 0.0)
                plsc.addupdate(asumacc.at[pl.ds(dloc * 16, 16)], w4m)
                for h in range(4):
                    whb = jnp.full((16,), wvec[h])
                    for q in range(4):
                        off = h * 64 + q * 16
                        plsc.addupdate(acc.at[pl.ds(dloc * 256 + off, 16)],
                                       whb * xlb[row, pl.ds(off, 16)])
        return 0
    lax.fori_loop(0, nchunks + 1, body, 0)

    pltpu.sync_copy(acc.at[pl.ds(0, RS * 256)],
                    outsum_hbm.at[pl.ds(w * RS * 256, RS * 256)])
    pltpu.sync_copy(asumacc.at[pl.ds(0, RS * 16)],
                    asum_hbm.at[pl.ds(w * RS * 16, RS * 16)])


@functools.partial(
    pl.kernel, mesh=_mesh, compiler_params=_CP,
    out_type=[
        jax.ShapeDtypeStruct((NP * 64,), jnp.float32),
        jax.ShapeDtypeStruct((NP,), jnp.float32),
    ],
    scratch_types=[
        pltpu.VMEM((RPAD * 64,), jnp.float32),
        pltpu.VMEM((RPAD,), jnp.float32),
        pltpu.VMEM((RS, 128), jnp.float32),
        pltpu.VMEM((32, 128), jnp.float32),
        pltpu.VMEM((32, 128), jnp.float32),
        pltpu.VMEM((2 * SCH,), jnp.int32),
        pltpu.VMEM((2 * SCH,), jnp.int32),
        pltpu.VMEM((2 * SCH,), jnp.int32),
        pltpu.VMEM((64,), jnp.float32),
        pltpu.VMEM((16,), jnp.int32),
        pltpu.SemaphoreType.DMA,
        pltpu.SemaphoreType.DMA,
    ],
)
def _sc_layer2(eidL, srcL, dlocL, counts, t2_hbm, ea2_hbm, att_hbm,
               outsum_hbm, asum_hbm,
               acc, asumacc, t2r, tsb, eab, eidS, srcS, dlocS, attv,
               cntb, s1, s3):
    w = _wid()
    lo = w * RS
    pltpu.sync_copy(att_hbm, attv)
    pltpu.sync_copy(counts.at[pl.ds(w * 16, 16)], cntb)
    cnt = cntb[pl.ds(0, 16)][0]
    pltpu.sync_copy(t2_hbm.at[pl.ds(lo, RS)], t2r)

    def z1(i, _):
        acc[pl.ds(i * 16, 16)] = jnp.zeros((16,), jnp.float32)
        return 0
    lax.fori_loop(0, RPAD * 64 // 16, z1, 0)

    def z2(i, _):
        asumacc[pl.ds(i * 16, 16)] = jnp.zeros((16,), jnp.float32)
        return 0
    lax.fori_loop(0, RPAD // 8 // 2, z2, 0)

    nchunks = ((cnt + SCH - 1) // SCH) * (SCH // 16)

    def body(k, _):
        @pl.when(k < nchunks)
        def _issue():
            ksup = k >> 5

            @pl.when((k & 31) == 0)
            def _ld():
                soff = w * CAP + ksup * SCH
                sl = (ksup & 1) * SCH
                pltpu.sync_copy(eidL.at[pl.ds(soff, SCH)], eidS.at[pl.ds(sl, SCH)])
                pltpu.sync_copy(srcL.at[pl.ds(soff, SCH)], srcS.at[pl.ds(sl, SCH)])
                pltpu.sync_copy(dlocL.at[pl.ds(soff, SCH)], dlocS.at[pl.ds(sl, SCH)])

            sl = (ksup & 1) * SCH + (k & 31) * 16
            srcv = srcS[pl.ds(sl, 16)]
            eidv = eidS[pl.ds(sl, 16)]
            b = (k & 1) * 16
            pltpu.async_copy(t2_hbm.at[srcv], tsb.at[pl.ds(b, 16)], s1)
            pltpu.async_copy(ea2_hbm.at[eidv], eab.at[pl.ds(b, 16)], s3)

        @pl.when(k > 0)
        def _comp():
            kp = k - 1
            b = (kp & 1) * 16
            pltpu.make_async_copy(t2_hbm.at[pl.ds(0, 16)],
                                  tsb.at[pl.ds(b, 16)], s1).wait()
            pltpu.make_async_copy(ea2_hbm.at[pl.ds(0, 16)],
                                  eab.at[pl.ds(b, 16)], s3).wait()
            slp = ((kp >> 5) & 1) * SCH + (kp & 31) * 16
            dlocv = dlocS[pl.ds(slp, 16)]
            lanes = _iota()
            for e in range(16):
                row = b + e
                dloc = dlocv[e]
                hacc = jnp.zeros((16,), jnp.float32)
                for q in range(4):
                    off = q * 16
                    z = (tsb[row, pl.ds(off, 16)] + t2r[dloc, pl.ds(off + 64, 16)]
                         + eab[row, pl.ds(off, 16)])
                    lr = jnp.maximum(z, 0.2 * z)
                    hacc = hacc + lr * attv[pl.ds(off, 16)]
                s = lax.reduce_sum(hacc, axes=(0,))
                whb = jnp.exp(jnp.full((16,), s))
                plsc.addupdate_scatter(asumacc, [jnp.full((16,), dloc)], whb,
                                       mask=lanes == 0)
                for q in range(4):
                    off = q * 16
                    plsc.addupdate(acc.at[pl.ds(dloc * 64 + off, 16)],
                                   whb * tsb[row, pl.ds(off, 16)])
        return 0
    lax.fori_loop(0, nchunks + 1, body, 0)

    pltpu.sync_copy(acc.at[pl.ds(0, RS * 64)],
                    outsum_hbm.at[pl.ds(w * RS * 64, RS * 64)])
    pltpu.sync_copy(asumacc.at[pl.ds(0, RS)], asum_hbm.at[pl.ds(w * RS, RS)])


def _tc_nodes(xp, Wl1, bl1, Wr1, br1):
    def body(x_ref, wl_ref, bl_ref, wr_ref, br_ref, ol_ref, or_ref):
        xb = x_ref[...]
        ol_ref[...] = jnp.dot(xb, wl_ref[...],
                              preferred_element_type=jnp.float32, precision=lax.Precision.HIGHEST) + bl_ref[...]
        or_ref[...] = jnp.dot(xb, wr_ref[...],
                              preferred_element_type=jnp.float32, precision=lax.Precision.HIGHEST) + br_ref[...]
    full = lambda s: pl.BlockSpec(s, lambda i: (0, 0))
    return pl.pallas_call(
        body,
        grid=(NP // 1024,),
        in_specs=[pl.BlockSpec((1024, 64), lambda i: (i, 0)),
                  full((64, 256)), full((1, 256)), full((64, 256)), full((1, 256))],
        out_specs=[pl.BlockSpec((1024, 256), lambda i: (i, 0)),
                   pl.BlockSpec((1024, 256), lambda i: (i, 0))],
        out_shape=[jax.ShapeDtypeStruct((NP, 256), jnp.float32),
                   jax.ShapeDtypeStruct((NP, 256), jnp.float32)],
    )(xp, Wl1, bl1.reshape(1, 256), Wr1, br1.reshape(1, 256))


def _tc_edges(eattr, We1, We2p):
    def body(e_ref, w1_ref, w2_ref, o1_ref, o2_ref):
        eb = e_ref[...]
        o1_ref[...] = jnp.dot(eb, w1_ref[...], preferred_element_type=jnp.float32, precision=lax.Precision.HIGHEST)
        o2_ref[...] = jnp.dot(eb, w2_ref[...], preferred_element_type=jnp.float32, precision=lax.Precision.HIGHEST)
    full = lambda s: pl.BlockSpec(s, lambda i: (0, 0))
    return pl.pallas_call(
        body,
        grid=(E // 8000,),
        in_specs=[pl.BlockSpec((8000, 16), lambda i: (i, 0)),
                  full((16, 256)), full((16, 128))],
        out_specs=[pl.BlockSpec((8000, 256), lambda i: (i, 0)),
                   pl.BlockSpec((8000, 128), lambda i: (i, 0))],
        out_shape=[jax.ShapeDtypeStruct((E, 256), jnp.float32),
                   jax.ShapeDtypeStruct((E, 128), jnp.float32)],
    )(eattr, We1, We2p)


def _tc_mid(outsum1, asum1, xl1, xr1, sumattr, deg, We1, We2, S, ST, b1,
            att1bc, att2col, Wl2, bl2, Wr2, br2):
    def body(os_ref, as_ref, xl_ref, xr_ref, sa_ref, dg_ref, we1_ref, we2_ref,
             s_ref, st_ref, b1_ref, a1_ref, a2_ref, wl2_ref, bl2_ref, wr2_ref,
             br2_ref, t2_ref, w2_ref):
        f32 = jnp.float32
        ma = sa_ref[...] / jnp.maximum(dg_ref[...], 1.0)
        la1 = jnp.dot(ma, we1_ref[...], preferred_element_type=f32, precision=lax.Precision.HIGHEST)
        xl = xl_ref[...]
        z = xl + xr_ref[...] + la1
        lr = jnp.maximum(z, 0.2 * z)
        logits = jnp.dot(lr * a1_ref[...], s_ref[...], preferred_element_type=f32, precision=lax.Precision.HIGHEST)
        wl1 = jnp.exp(logits)
        wl1b = jnp.dot(wl1, st_ref[...], preferred_element_type=f32, precision=lax.Precision.HIGHEST)
        num = os_ref[...] + wl1b * xl
        den = jnp.dot(as_ref[...][:, :4] + wl1, st_ref[...],
                      preferred_element_type=f32, precision=lax.Precision.HIGHEST)
        h1 = jnp.maximum(num / den + b1_ref[...], 0.0)
        xl2 = jnp.dot(h1, wl2_ref[...], preferred_element_type=f32, precision=lax.Precision.HIGHEST) + bl2_ref[...]
        xr2 = jnp.dot(h1, wr2_ref[...], preferred_element_type=f32, precision=lax.Precision.HIGHEST) + br2_ref[...]
        t2_ref[...] = jnp.concatenate([xl2, xr2], axis=1)
        la2 = jnp.dot(ma, we2_ref[...], preferred_element_type=f32, precision=lax.Precision.HIGHEST)
        z2 = xl2 + xr2 + la2
        lr2 = jnp.maximum(z2, 0.2 * z2)
        w2_ref[...] = jnp.exp(jnp.dot(lr2, a2_ref[...], preferred_element_type=f32, precision=lax.Precision.HIGHEST))
    full = lambda s: pl.BlockSpec(s, lambda i: (0, 0))
    blk = lambda s: pl.BlockSpec(s, lambda i: (i, 0))
    return pl.pallas_call(
        body,
        grid=(NP // 1024,),
        in_specs=[blk((1024, 256)), blk((1024, 16)), blk((1024, 256)),
                  blk((1024, 256)), blk((1024, 16)), blk((1024, 1)),
                  full((16, 256)), full((16, 64)), full((256, 4)),
                  full((4, 256)), full((1, 256)), full((1, 256)),
                  full((64, 1)), full((256, 64)), full((1, 64)),
                  full((256, 64)), full((1, 64))],
        out_specs=[blk((1024, 128)), blk((1024, 1))],
        out_shape=[jax.ShapeDtypeStruct((NP, 128), jnp.float32),
                   jax.ShapeDtypeStruct((NP, 1), jnp.float32)],
    )(outsum1, asum1, xl1, xr1, sumattr, deg, We1, We2, S, ST, b1, att1bc,
      att2col, Wl2, bl2, Wr2, br2)


def _tc_head(outsum2, asum2, wl2, T2, b2, Wh1, bh1, Wh2, bh2):
    def body(os_ref, as_ref, w2_ref, t2_ref, b2_ref, wh1_ref, bh1_ref,
             wh2_ref, bh2_ref, y_ref):
        f32 = jnp.float32
        xl2 = t2_ref[...][:, :64]
        w2 = w2_ref[...]
        out2 = (os_ref[...] + w2 * xl2) / (as_ref[...] + w2) + b2_ref[...]
        hh = jnp.maximum(jnp.dot(out2, wh1_ref[...], preferred_element_type=f32, precision=lax.Precision.HIGHEST)
                         + bh1_ref[...], 0.0)
        y_ref[...] = jnp.dot(hh, wh2_ref[...],
                             preferred_element_type=f32, precision=lax.Precision.HIGHEST) + bh2_ref[...]
    full = lambda s: pl.BlockSpec(s, lambda i: (0, 0))
    blk = lambda s: pl.BlockSpec(s, lambda i: (i, 0))
    return pl.pallas_call(
        body,
        grid=(NP // 1024,),
        in_specs=[blk((1024, 64)), blk((1024, 1)), blk((1024, 1)),
                  blk((1024, 128)), full((1, 64)), full((64, 64)),
                  full((1, 64)), full((64, 2)), full((1, 2))],
        out_specs=blk((1024, 2)),
        out_shape=jax.ShapeDtypeStruct((NP, 2), jnp.float32),
    )(outsum2, asum2, wl2, T2, b2.reshape(1, 64), Wh1, bh1.reshape(1, 64),
      Wh2, bh2.reshape(1, 2))


def kernel(x, edge_index, edge_attr, Wl1, bl1, Wr1, br1, We1, att1, b1,
           Wl2, bl2, Wr2, br2, We2, att2, b2, Wh1, bh1, Wh2, bh2):
    src = edge_index[0]
    dst = edge_index[1]
    xp = jnp.pad(x, ((0, NP - N), (0, 0)))
    ea128 = edge_attr.reshape(E // 8, 128)
    We2p = jnp.pad(We2, ((0, 0), (0, 64)))

    eidL, srcL, dlocL, counts, deg, sumattr_f = _sc_route(src, dst, ea128)
    sumattr = sumattr_f.reshape(NP, 16)

    xl1, xr1 = _tc_nodes(xp, Wl1, bl1, Wr1, br1)
    ea1, ea2p = _tc_edges(edge_attr, We1, We2p)

    outsum1_f, asum1_f = _sc_layer1(eidL, srcL, dlocL, counts, xl1, xr1, ea1,
                                    att1.reshape(256))
    outsum1 = outsum1_f.reshape(NP, 256)
    asum1 = asum1_f.reshape(NP, 16)

    S = jnp.kron(jnp.eye(4, dtype=jnp.float32), jnp.ones((64, 1), jnp.float32))
    T2, wl2 = _tc_mid(outsum1, asum1, xl1, xr1, sumattr, deg.reshape(NP, 1),
                      We1, We2, S, S.T, b1.reshape(1, 256),
                      att1.reshape(1, 256), att2.reshape(64, 1),
                      Wl2, bl2.reshape(1, 64), Wr2, br2.reshape(1, 64))

    outsum2_f, asum2_f = _sc_layer2(eidL, srcL, dlocL, counts, T2, ea2p,
                                    att2.reshape(64))
    outsum2 = outsum2_f.reshape(NP, 64)
    asum2 = asum2_f.reshape(NP, 1)

    y = _tc_head(outsum2, asum2, wl2, T2, b2, Wh1, bh1, Wh2, bh2)
    return y[:N]


# speedup vs baseline: 11.9382x; 1.0702x over previous
import functools
import jax
import jax.numpy as jnp
from jax import lax
from jax.experimental import pallas as pl
from jax.experimental.pallas import tpu as pltpu
from jax.experimental.pallas import tpu_sc as plsc

N = 10000
E = 320000
NP = 10240
NW = 32
RS = 320
RPAD = 328
CAP = 12800
CK = 2000
SCH = 512

_mesh = plsc.VectorSubcoreMesh(core_axis_name="c", subcore_axis_name="s")
_CP = pltpu.CompilerParams(needs_layout_passes=False)


def _wid():
    return lax.axis_index("s") * 2 + lax.axis_index("c")


def _iota():
    return lax.iota(jnp.int32, 16)


@functools.partial(
    pl.kernel, mesh=_mesh, compiler_params=_CP,
    out_type=[
        jax.ShapeDtypeStruct((NW * CAP,), jnp.int32),
        jax.ShapeDtypeStruct((NW * CAP,), jnp.int32),
        jax.ShapeDtypeStruct((NW * CAP,), jnp.int32),
        jax.ShapeDtypeStruct((NW * 16,), jnp.int32),
        jax.ShapeDtypeStruct((NP,), jnp.float32),
        jax.ShapeDtypeStruct((NP * 16,), jnp.float32),
    ],
    scratch_types=[
        pltpu.VMEM((CK,), jnp.int32),
        pltpu.VMEM((CK,), jnp.int32),
        pltpu.VMEM((CAP,), jnp.int32),
        pltpu.VMEM((CAP,), jnp.int32),
        pltpu.VMEM((CAP,), jnp.int32),
        pltpu.VMEM((64, 128), jnp.float32),
        pltpu.VMEM((328,), jnp.float32),
        pltpu.VMEM((5248,), jnp.float32),
        pltpu.VMEM((16,), jnp.int32),
        pltpu.SemaphoreType.DMA,
    ],
)
def _sc_route(src_hbm, dst_hbm, ea128_hbm,
              eidL, srcL, dlocL, counts, deg_hbm, sumattr_hbm,
              srcc, dstc, eidS, srcS, dlocS, eatb, degacc, sumacc,
              cntst, sem):
    w = _wid()
    lo = w * RS

    def pre(g, _):
        base = g * 16 + _iota()
        eidS[pl.ds(g * 16, 16)] = (base * 7919) % E
        srcS[pl.ds(g * 16, 16)] = (base * 9973) % N
        dlocS[pl.ds(g * 16, 16)] = jnp.full((16,), RS, jnp.int32)
        return 0
    lax.fori_loop(0, CAP // 16, pre, 0)

    def zero_deg(i, _):
        degacc[pl.ds(i * 16, 16)] = jnp.zeros((16,), jnp.float32)
        return 0
    lax.fori_loop(0, 328 // 16, zero_deg, 0)

    def zero_sa(i, _):
        sumacc[pl.ds(i * 16, 16)] = jnp.zeros((16,), jnp.float32)
        return 0
    lax.fori_loop(0, 5248 // 16, zero_sa, 0)

    def chunk(k, cnt):
        pltpu.sync_copy(src_hbm.at[pl.ds(k * CK, CK)], srcc)
        pltpu.sync_copy(dst_hbm.at[pl.ds(k * CK, CK)], dstc)

        def grp(g, cnt):
            sv = srcc[pl.ds(g * 16, 16)]
            dv = dstc[pl.ds(g * 16, 16)]
            ev = k * CK + g * 16 + _iota()
            m = jnp.logical_and(dv >= lo, dv < lo + RS)
            cs = plsc.cumsum(m.astype(jnp.int32))
            pos = cnt + cs - 1
            plsc.store_scatter(eidS, [pos], ev, mask=m)
            plsc.store_scatter(srcS, [pos], sv, mask=m)
            plsc.store_scatter(dlocS, [pos], dv - lo, mask=m)
            return cnt + cs[15]
        return lax.fori_loop(0, CK // 16, grp, cnt)

    cnt = lax.fori_loop(0, E // CK, chunk, jnp.int32(0))

    pltpu.sync_copy(eidS, eidL.at[pl.ds(w * CAP, CAP)])
    pltpu.sync_copy(srcS, srcL.at[pl.ds(w * CAP, CAP)])
    pltpu.sync_copy(dlocS, dlocL.at[pl.ds(w * CAP, CAP)])
    cntst[...] = jnp.full((16,), cnt, jnp.int32)
    pltpu.sync_copy(cntst, counts.at[pl.ds(w * 16, 16)])

    trips = (cnt + 31) // 32

    def seg_issue(k):
        b = (k & 1) * 32
        for g in range(2):
            ev = eidS[pl.ds(k * 32 + g * 16, 16)]
            pltpu.async_copy(ea128_hbm.at[ev >> 3],
                             eatb.at[pl.ds(b + g * 16, 16)], sem)

    def seg(k, _):
        @pl.when(k < trips)
        def _():
            seg_issue(k)

        @pl.when(k > 0)
        def _():
            kp = k - 1
            b = (kp & 1) * 32
            for g in range(2):
                pltpu.make_async_copy(ea128_hbm.at[pl.ds(0, 16)],
                                      eatb.at[pl.ds(b + g * 16, 16)], sem).wait()
            for g in range(2):
                ev = eidS[pl.ds(kp * 32 + g * 16, 16)]
                dl = dlocS[pl.ds(kp * 32 + g * 16, 16)]
                colbase = (ev & 7) * 16
                plsc.addupdate_scatter(degacc, [dl], jnp.ones((16,), jnp.float32))
                for e in range(16):
                    vals = eatb[b + g * 16 + e, pl.ds(colbase[e], 16)]
                    plsc.addupdate(sumacc.at[pl.ds(dl[e] * 16, 16)], vals)
        return 0
    lax.fori_loop(0, trips + 1, seg, 0)

    pltpu.sync_copy(degacc.at[pl.ds(0, RS)], deg_hbm.at[pl.ds(w * RS, RS)])
    pltpu.sync_copy(sumacc.at[pl.ds(0, RS * 16)],
                    sumattr_hbm.at[pl.ds(w * RS * 16, RS * 16)])


@functools.partial(
    pl.kernel, mesh=_mesh, compiler_params=_CP,
    out_type=[
        jax.ShapeDtypeStruct((NP * 256,), jnp.float32),
        jax.ShapeDtypeStruct((NP * 16,), jnp.float32),
    ],
    scratch_types=[
        pltpu.VMEM((RPAD * 256,), jnp.float32),
        pltpu.VMEM((RPAD * 16,), jnp.float32),
        pltpu.VMEM((32, 256), jnp.float32),
        pltpu.VMEM((32, 256), jnp.float32),
        pltpu.VMEM((32, 256), jnp.float32),
        pltpu.VMEM((2 * SCH,), jnp.int32),
        pltpu.VMEM((2 * SCH,), jnp.int32),
        pltpu.VMEM((2 * SCH,), jnp.int32),
        pltpu.VMEM((256,), jnp.float32),
        pltpu.VMEM((16,), jnp.int32),
        pltpu.SemaphoreType.DMA,
        pltpu.SemaphoreType.DMA,
        pltpu.SemaphoreType.DMA,
    ],
)
def _sc_layer1(eidL, srcL, dlocL, counts, xl1_hbm, xr1_hbm, ea1_hbm, att_hbm,
               outsum_hbm, asum_hbm,
               acc, asumacc, xlb, xrb, eab, eidS, srcS, dlocS, attv,
               cntb, s1, s2, s3):
    w = _wid()
    lo = w * RS
    pltpu.sync_copy(att_hbm, attv)
    pltpu.sync_copy(counts.at[pl.ds(w * 16, 16)], cntb)
    cnt = cntb[pl.ds(0, 16)][0]

    def z1(i, _):
        acc[pl.ds(i * 16, 16)] = jnp.zeros((16,), jnp.float32)
        return 0
    lax.fori_loop(0, RPAD * 256 // 16, z1, 0)

    def z2(i, _):
        asumacc[pl.ds(i * 16, 16)] = jnp.zeros((16,), jnp.float32)
        return 0
    lax.fori_loop(0, RPAD, z2, 0)

    nchunks = ((cnt + SCH - 1) // SCH) * (SCH // 16)

    def body(k, _):
        @pl.when(k < nchunks)
        def _issue():
            ksup = k >> 5

            @pl.when((k & 31) == 0)
            def _ld():
                soff = w * CAP + ksup * SCH
                sl = (ksup & 1) * SCH
                pltpu.sync_copy(eidL.at[pl.ds(soff, SCH)], eidS.at[pl.ds(sl, SCH)])
                pltpu.sync_copy(srcL.at[pl.ds(soff, SCH)], srcS.at[pl.ds(sl, SCH)])
                pltpu.sync_copy(dlocL.at[pl.ds(soff, SCH)], dlocS.at[pl.ds(sl, SCH)])

            sl = (ksup & 1) * SCH + (k & 31) * 16
            srcv = srcS[pl.ds(sl, 16)]
            dstv = dlocS[pl.ds(sl, 16)] + lo
            eidv = eidS[pl.ds(sl, 16)]
            b = (k & 1) * 16
            pltpu.async_copy(xl1_hbm.at[srcv], xlb.at[pl.ds(b, 16)], s1)
            pltpu.async_copy(xr1_hbm.at[dstv], xrb.at[pl.ds(b, 16)], s2)
            pltpu.async_copy(ea1_hbm.at[eidv], eab.at[pl.ds(b, 16)], s3)

        @pl.when(k > 0)
        def _comp():
            kp = k - 1
            b = (kp & 1) * 16
            pltpu.make_async_copy(xl1_hbm.at[pl.ds(0, 16)],
                                  xlb.at[pl.ds(b, 16)], s1).wait()
            pltpu.make_async_copy(xr1_hbm.at[pl.ds(0, 16)],
                                  xrb.at[pl.ds(b, 16)], s2).wait()
            pltpu.make_async_copy(ea1_hbm.at[pl.ds(0, 16)],
                                  eab.at[pl.ds(b, 16)], s3).wait()
            slp = ((kp >> 5) & 1) * SCH + (kp & 31) * 16
            dlocv = dlocS[pl.ds(slp, 16)]
            lanes = _iota()
            for e in range(16):
                row = b + e
                dloc = dlocv[e]
                w4m = jnp.zeros((16,), jnp.float32)
                for h in range(4):
                    hacc = jnp.zeros((16,), jnp.float32)
                    xls = []
                    for q in range(4):
                        off = h * 64 + q * 16
                        xlv = xlb[row, pl.ds(off, 16)]
                        xls.append(xlv)
                        z = xlv + xrb[row, pl.ds(off, 16)] + eab[row, pl.ds(off, 16)]
                        lr = jnp.maximum(z, 0.2 * z)
                        hacc = hacc + lr * attv[pl.ds(off, 16)]
                    s = lax.reduce_sum(hacc, axes=(0,))
                    whb = jnp.exp(jnp.full((16,), s))
                    w4m = w4m + jnp.where(lanes == h, whb, 0.0)
                    for q in range(4):
                        off = h * 64 + q * 16
                        plsc.addupdate(acc.at[pl.ds(dloc * 256 + off, 16)],
                                       whb * xls[q])
                plsc.addupdate(asumacc.at[pl.ds(dloc * 16, 16)], w4m)
        return 0
    lax.fori_loop(0, nchunks + 1, body, 0)

    pltpu.sync_copy(acc.at[pl.ds(0, RS * 256)],
                    outsum_hbm.at[pl.ds(w * RS * 256, RS * 256)])
    pltpu.sync_copy(asumacc.at[pl.ds(0, RS * 16)],
                    asum_hbm.at[pl.ds(w * RS * 16, RS * 16)])


@functools.partial(
    pl.kernel, mesh=_mesh, compiler_params=_CP,
    out_type=[
        jax.ShapeDtypeStruct((NP * 64,), jnp.float32),
        jax.ShapeDtypeStruct((NP,), jnp.float32),
    ],
    scratch_types=[
        pltpu.VMEM((RPAD * 64,), jnp.float32),
        pltpu.VMEM((RPAD,), jnp.float32),
        pltpu.VMEM((RS, 128), jnp.float32),
        pltpu.VMEM((32, 128), jnp.float32),
        pltpu.VMEM((32, 128), jnp.float32),
        pltpu.VMEM((2 * SCH,), jnp.int32),
        pltpu.VMEM((2 * SCH,), jnp.int32),
        pltpu.VMEM((2 * SCH,), jnp.int32),
        pltpu.VMEM((64,), jnp.float32),
        pltpu.VMEM((16,), jnp.int32),
        pltpu.SemaphoreType.DMA,
        pltpu.SemaphoreType.DMA,
    ],
)
def _sc_layer2(eidL, srcL, dlocL, counts, t2_hbm, ea2_hbm, att_hbm,
               outsum_hbm, asum_hbm,
               acc, asumacc, t2r, tsb, eab, eidS, srcS, dlocS, attv,
               cntb, s1, s3):
    w = _wid()
    lo = w * RS
    pltpu.sync_copy(att_hbm, attv)
    pltpu.sync_copy(counts.at[pl.ds(w * 16, 16)], cntb)
    cnt = cntb[pl.ds(0, 16)][0]
    pltpu.sync_copy(t2_hbm.at[pl.ds(lo, RS)], t2r)

    def z1(i, _):
        acc[pl.ds(i * 16, 16)] = jnp.zeros((16,), jnp.float32)
        return 0
    lax.fori_loop(0, RPAD * 64 // 16, z1, 0)

    def z2(i, _):
        asumacc[pl.ds(i * 16, 16)] = jnp.zeros((16,), jnp.float32)
        return 0
    lax.fori_loop(0, RPAD // 8 // 2, z2, 0)

    nchunks = ((cnt + SCH - 1) // SCH) * (SCH // 16)

    def body(k, _):
        @pl.when(k < nchunks)
        def _issue():
            ksup = k >> 5

            @pl.when((k & 31) == 0)
            def _ld():
                soff = w * CAP + ksup * SCH
                sl = (ksup & 1) * SCH
                pltpu.sync_copy(eidL.at[pl.ds(soff, SCH)], eidS.at[pl.ds(sl, SCH)])
                pltpu.sync_copy(srcL.at[pl.ds(soff, SCH)], srcS.at[pl.ds(sl, SCH)])
                pltpu.sync_copy(dlocL.at[pl.ds(soff, SCH)], dlocS.at[pl.ds(sl, SCH)])

            sl = (ksup & 1) * SCH + (k & 31) * 16
            srcv = srcS[pl.ds(sl, 16)]
            eidv = eidS[pl.ds(sl, 16)]
            b = (k & 1) * 16
            pltpu.async_copy(t2_hbm.at[srcv], tsb.at[pl.ds(b, 16)], s1)
            pltpu.async_copy(ea2_hbm.at[eidv], eab.at[pl.ds(b, 16)], s3)

        @pl.when(k > 0)
        def _comp():
            kp = k - 1
            b = (kp & 1) * 16
            pltpu.make_async_copy(t2_hbm.at[pl.ds(0, 16)],
                                  tsb.at[pl.ds(b, 16)], s1).wait()
            pltpu.make_async_copy(ea2_hbm.at[pl.ds(0, 16)],
                                  eab.at[pl.ds(b, 16)], s3).wait()
            slp = ((kp >> 5) & 1) * SCH + (kp & 31) * 16
            dlocv = dlocS[pl.ds(slp, 16)]
            lanes = _iota()
            for e in range(16):
                row = b + e
                dloc = dlocv[e]
                hacc = jnp.zeros((16,), jnp.float32)
                xls = []
                for q in range(4):
                    off = q * 16
                    xlv = tsb[row, pl.ds(off, 16)]
                    xls.append(xlv)
                    z = xlv + t2r[dloc, pl.ds(off + 64, 16)] + eab[row, pl.ds(off, 16)]
                    lr = jnp.maximum(z, 0.2 * z)
                    hacc = hacc + lr * attv[pl.ds(off, 16)]
                s = lax.reduce_sum(hacc, axes=(0,))
                whb = jnp.exp(jnp.full((16,), s))
                plsc.addupdate_scatter(asumacc, [jnp.full((16,), dloc)], whb,
                                       mask=lanes == 0)
                for q in range(4):
                    off = q * 16
                    plsc.addupdate(acc.at[pl.ds(dloc * 64 + off, 16)],
                                   whb * xls[q])
        return 0
    lax.fori_loop(0, nchunks + 1, body, 0)

    pltpu.sync_copy(acc.at[pl.ds(0, RS * 64)],
                    outsum_hbm.at[pl.ds(w * RS * 64, RS * 64)])
    pltpu.sync_copy(asumacc.at[pl.ds(0, RS)], asum_hbm.at[pl.ds(w * RS, RS)])


def _tc_nodes(xp, Wl1, bl1, Wr1, br1):
    def body(x_ref, wl_ref, bl_ref, wr_ref, br_ref, ol_ref, or_ref):
        xb = x_ref[...]
        ol_ref[...] = jnp.dot(xb, wl_ref[...],
                              preferred_element_type=jnp.float32, precision=lax.Precision.HIGHEST) + bl_ref[...]
        or_ref[...] = jnp.dot(xb, wr_ref[...],
                              preferred_element_type=jnp.float32, precision=lax.Precision.HIGHEST) + br_ref[...]
    full = lambda s: pl.BlockSpec(s, lambda i: (0, 0))
    return pl.pallas_call(
        body,
        grid=(NP // 1024,),
        in_specs=[pl.BlockSpec((1024, 64), lambda i: (i, 0)),
                  full((64, 256)), full((1, 256)), full((64, 256)), full((1, 256))],
        out_specs=[pl.BlockSpec((1024, 256), lambda i: (i, 0)),
                   pl.BlockSpec((1024, 256), lambda i: (i, 0))],
        out_shape=[jax.ShapeDtypeStruct((NP, 256), jnp.float32),
                   jax.ShapeDtypeStruct((NP, 256), jnp.float32)],
    )(xp, Wl1, bl1.reshape(1, 256), Wr1, br1.reshape(1, 256))


def _tc_edges(eattr, We1, We2p):
    def body(e_ref, w1_ref, w2_ref, o1_ref, o2_ref):
        eb = e_ref[...]
        o1_ref[...] = jnp.dot(eb, w1_ref[...], preferred_element_type=jnp.float32, precision=lax.Precision.HIGHEST)
        o2_ref[...] = jnp.dot(eb, w2_ref[...], preferred_element_type=jnp.float32, precision=lax.Precision.HIGHEST)
    full = lambda s: pl.BlockSpec(s, lambda i: (0, 0))
    return pl.pallas_call(
        body,
        grid=(E // 8000,),
        in_specs=[pl.BlockSpec((8000, 16), lambda i: (i, 0)),
                  full((16, 256)), full((16, 128))],
        out_specs=[pl.BlockSpec((8000, 256), lambda i: (i, 0)),
                   pl.BlockSpec((8000, 128), lambda i: (i, 0))],
        out_shape=[jax.ShapeDtypeStruct((E, 256), jnp.float32),
                   jax.ShapeDtypeStruct((E, 128), jnp.float32)],
    )(eattr, We1, We2p)


def _tc_mid(outsum1, asum1, xl1, xr1, sumattr, deg, We1, We2, S, ST, b1,
            att1bc, att2col, Wl2, bl2, Wr2, br2):
    def body(os_ref, as_ref, xl_ref, xr_ref, sa_ref, dg_ref, we1_ref, we2_ref,
             s_ref, st_ref, b1_ref, a1_ref, a2_ref, wl2_ref, bl2_ref, wr2_ref,
             br2_ref, t2_ref, w2_ref):
        f32 = jnp.float32
        ma = sa_ref[...] / jnp.maximum(dg_ref[...], 1.0)
        la1 = jnp.dot(ma, we1_ref[...], preferred_element_type=f32, precision=lax.Precision.HIGHEST)
        xl = xl_ref[...]
        z = xl + xr_ref[...] + la1
        lr = jnp.maximum(z, 0.2 * z)
        logits = jnp.dot(lr * a1_ref[...], s_ref[...], preferred_element_type=f32, precision=lax.Precision.HIGHEST)
        wl1 = jnp.exp(logits)
        wl1b = jnp.dot(wl1, st_ref[...], preferred_element_type=f32, precision=lax.Precision.HIGHEST)
        num = os_ref[...] + wl1b * xl
        den = jnp.dot(as_ref[...][:, :4] + wl1, st_ref[...],
                      preferred_element_type=f32, precision=lax.Precision.HIGHEST)
        h1 = jnp.maximum(num / den + b1_ref[...], 0.0)
        xl2 = jnp.dot(h1, wl2_ref[...], preferred_element_type=f32, precision=lax.Precision.HIGHEST) + bl2_ref[...]
        xr2 = jnp.dot(h1, wr2_ref[...], preferred_element_type=f32, precision=lax.Precision.HIGHEST) + br2_ref[...]
        t2_ref[...] = jnp.concatenate([xl2, xr2], axis=1)
        la2 = jnp.dot(ma, we2_ref[...], preferred_element_type=f32, precision=lax.Precision.HIGHEST)
        z2 = xl2 + xr2 + la2
        lr2 = jnp.maximum(z2, 0.2 * z2)
        w2_ref[...] = jnp.exp(jnp.dot(lr2, a2_ref[...], preferred_element_type=f32, precision=lax.Precision.HIGHEST))
    full = lambda s: pl.BlockSpec(s, lambda i: (0, 0))
    blk = lambda s: pl.BlockSpec(s, lambda i: (i, 0))
    return pl.pallas_call(
        body,
        grid=(NP // 1024,),
        in_specs=[blk((1024, 256)), blk((1024, 16)), blk((1024, 256)),
                  blk((1024, 256)), blk((1024, 16)), blk((1024, 1)),
                  full((16, 256)), full((16, 64)), full((256, 4)),
                  full((4, 256)), full((1, 256)), full((1, 256)),
                  full((64, 1)), full((256, 64)), full((1, 64)),
                  full((256, 64)), full((1, 64))],
        out_specs=[blk((1024, 128)), blk((1024, 1))],
        out_shape=[jax.ShapeDtypeStruct((NP, 128), jnp.float32),
                   jax.ShapeDtypeStruct((NP, 1), jnp.float32)],
    )(outsum1, asum1, xl1, xr1, sumattr, deg, We1, We2, S, ST, b1, att1bc,
      att2col, Wl2, bl2, Wr2, br2)


def _tc_head(outsum2, asum2, wl2, T2, b2, Wh1, bh1, Wh2, bh2):
    def body(os_ref, as_ref, w2_ref, t2_ref, b2_ref, wh1_ref, bh1_ref,
             wh2_ref, bh2_ref, y_ref):
        f32 = jnp.float32
        xl2 = t2_ref[...][:, :64]
        w2 = w2_ref[...]
        out2 = (os_ref[...] + w2 * xl2) / (as_ref[...] + w2) + b2_ref[...]
        hh = jnp.maximum(jnp.dot(out2, wh1_ref[...], preferred_element_type=f32, precision=lax.Precision.HIGHEST)
                         + bh1_ref[...], 0.0)
        y_ref[...] = jnp.dot(hh, wh2_ref[...],
                             preferred_element_type=f32, precision=lax.Precision.HIGHEST) + bh2_ref[...]
    full = lambda s: pl.BlockSpec(s, lambda i: (0, 0))
    blk = lambda s: pl.BlockSpec(s, lambda i: (i, 0))
    return pl.pallas_call(
        body,
        grid=(NP // 1024,),
        in_specs=[blk((1024, 64)), blk((1024, 1)), blk((1024, 1)),
                  blk((1024, 128)), full((1, 64)), full((64, 64)),
                  full((1, 64)), full((64, 2)), full((1, 2))],
        out_specs=blk((1024, 2)),
        out_shape=jax.ShapeDtypeStruct((NP, 2), jnp.float32),
    )(outsum2, asum2, wl2, T2, b2.reshape(1, 64), Wh1, bh1.reshape(1, 64),
      Wh2, bh2.reshape(1, 2))


def kernel(x, edge_index, edge_attr, Wl1, bl1, Wr1, br1, We1, att1, b1,
           Wl2, bl2, Wr2, br2, We2, att2, b2, Wh1, bh1, Wh2, bh2):
    src = edge_index[0]
    dst = edge_index[1]
    xp = jnp.pad(x, ((0, NP - N), (0, 0)))
    ea128 = edge_attr.reshape(E // 8, 128)
    We2p = jnp.pad(We2, ((0, 0), (0, 64)))

    eidL, srcL, dlocL, counts, deg, sumattr_f = _sc_route(src, dst, ea128)
    sumattr = sumattr_f.reshape(NP, 16)

    xl1, xr1 = _tc_nodes(xp, Wl1, bl1, Wr1, br1)
    ea1, ea2p = _tc_edges(edge_attr, We1, We2p)

    outsum1_f, asum1_f = _sc_layer1(eidL, srcL, dlocL, counts, xl1, xr1, ea1,
                                    att1.reshape(256))
    outsum1 = outsum1_f.reshape(NP, 256)
    asum1 = asum1_f.reshape(NP, 16)

    S = jnp.kron(jnp.eye(4, dtype=jnp.float32), jnp.ones((64, 1), jnp.float32))
    T2, wl2 = _tc_mid(outsum1, asum1, xl1, xr1, sumattr, deg.reshape(NP, 1),
                      We1, We2, S, S.T, b1.reshape(1, 256),
                      att1.reshape(1, 256), att2.reshape(64, 1),
                      Wl2, bl2.reshape(1, 64), Wr2, br2.reshape(1, 64))

    outsum2_f, asum2_f = _sc_layer2(eidL, srcL, dlocL, counts, T2, ea2p,
                                    att2.reshape(64))
    outsum2 = outsum2_f.reshape(NP, 64)
    asum2 = asum2_f.reshape(NP, 1)

    y = _tc_head(outsum2, asum2, wl2, T2, b2, Wh1, bh1, Wh2, bh2)
    return y[:N]


# speedup vs baseline: 12.4072x; 1.0393x over previous
import functools
import jax
import jax.numpy as jnp
from jax import lax
from jax.experimental import pallas as pl
from jax.experimental.pallas import tpu as pltpu
from jax.experimental.pallas import tpu_sc as plsc

N = 10000
E = 320000
NP = 10240
NW = 32
RS = 320
RPAD = 328
CAP = 12800
CK = 8000
SCH = 512

_mesh = plsc.VectorSubcoreMesh(core_axis_name="c", subcore_axis_name="s")
_CP = pltpu.CompilerParams(needs_layout_passes=False)


def _wid():
    return lax.axis_index("s") * 2 + lax.axis_index("c")


def _iota():
    return lax.iota(jnp.int32, 16)


@functools.partial(
    pl.kernel, mesh=_mesh, compiler_params=_CP,
    out_type=[
        jax.ShapeDtypeStruct((NW * CAP,), jnp.int32),
        jax.ShapeDtypeStruct((NW * CAP,), jnp.int32),
        jax.ShapeDtypeStruct((NW * CAP,), jnp.int32),
        jax.ShapeDtypeStruct((NW * 16,), jnp.int32),
        jax.ShapeDtypeStruct((NP,), jnp.float32),
        jax.ShapeDtypeStruct((NP * 16,), jnp.float32),
    ],
    scratch_types=[
        pltpu.VMEM((CK,), jnp.int32),
        pltpu.VMEM((CK,), jnp.int32),
        pltpu.VMEM((CAP,), jnp.int32),
        pltpu.VMEM((CAP,), jnp.int32),
        pltpu.VMEM((CAP,), jnp.int32),
        pltpu.VMEM((64, 128), jnp.float32),
        pltpu.VMEM((328,), jnp.float32),
        pltpu.VMEM((5248,), jnp.float32),
        pltpu.VMEM((16,), jnp.int32),
        pltpu.SemaphoreType.DMA,
    ],
)
def _sc_route(src_hbm, dst_hbm, ea128_hbm,
              eidL, srcL, dlocL, counts, deg_hbm, sumattr_hbm,
              srcc, dstc, eidS, srcS, dlocS, eatb, degacc, sumacc,
              cntst, sem):
    w = _wid()
    lo = w * RS

    def pre(g, _):
        base = g * 16 + _iota()
        eidS[pl.ds(g * 16, 16)] = (base * 7919) % E
        srcS[pl.ds(g * 16, 16)] = (base * 9973) % N
        dlocS[pl.ds(g * 16, 16)] = jnp.full((16,), RS, jnp.int32)
        return 0
    lax.fori_loop(0, CAP // 16, pre, 0)

    def zero_deg(i, _):
        degacc[pl.ds(i * 16, 16)] = jnp.zeros((16,), jnp.float32)
        return 0
    lax.fori_loop(0, 328 // 16, zero_deg, 0)

    def zero_sa(i, _):
        sumacc[pl.ds(i * 16, 16)] = jnp.zeros((16,), jnp.float32)
        return 0
    lax.fori_loop(0, 5248 // 16, zero_sa, 0)

    def chunk(k, cnt):
        pltpu.sync_copy(src_hbm.at[pl.ds(k * CK, CK)], srcc)
        pltpu.sync_copy(dst_hbm.at[pl.ds(k * CK, CK)], dstc)

        def grp(g, cnt):
            sv = srcc[pl.ds(g * 16, 16)]
            dv = dstc[pl.ds(g * 16, 16)]
            ev = k * CK + g * 16 + _iota()
            m = jnp.logical_and(dv >= lo, dv < lo + RS)
            cs = plsc.cumsum(m.astype(jnp.int32))
            pos = cnt + cs - 1
            plsc.store_scatter(eidS, [pos], ev, mask=m)
            plsc.store_scatter(srcS, [pos], sv, mask=m)
            plsc.store_scatter(dlocS, [pos], dv - lo, mask=m)
            return cnt + cs[15]
        return lax.fori_loop(0, CK // 16, grp, cnt)

    cnt = lax.fori_loop(0, E // CK, chunk, jnp.int32(0))

    pltpu.sync_copy(eidS, eidL.at[pl.ds(w * CAP, CAP)])
    pltpu.sync_copy(srcS, srcL.at[pl.ds(w * CAP, CAP)])
    pltpu.sync_copy(dlocS, dlocL.at[pl.ds(w * CAP, CAP)])
    cntst[...] = jnp.full((16,), cnt, jnp.int32)
    pltpu.sync_copy(cntst, counts.at[pl.ds(w * 16, 16)])

    trips = (cnt + 31) // 32

    def seg_issue(k):
        b = (k & 1) * 32
        for g in range(2):
            ev = eidS[pl.ds(k * 32 + g * 16, 16)]
            pltpu.async_copy(ea128_hbm.at[ev >> 3],
                             eatb.at[pl.ds(b + g * 16, 16)], sem)

    def seg(k, _):
        @pl.when(k < trips)
        def _():
            seg_issue(k)

        @pl.when(k > 0)
        def _():
            kp = k - 1
            b = (kp & 1) * 32
            for g in range(2):
                pltpu.make_async_copy(ea128_hbm.at[pl.ds(0, 16)],
                                      eatb.at[pl.ds(b + g * 16, 16)], sem).wait()
            for g in range(2):
                ev = eidS[pl.ds(kp * 32 + g * 16, 16)]
                dl = dlocS[pl.ds(kp * 32 + g * 16, 16)]
                colbase = (ev & 7) * 16
                plsc.addupdate_scatter(degacc, [dl], jnp.ones((16,), jnp.float32))
                for e in range(16):
                    vals = eatb[b + g * 16 + e, pl.ds(colbase[e], 16)]
                    plsc.addupdate(sumacc.at[pl.ds(dl[e] * 16, 16)], vals)
        return 0
    lax.fori_loop(0, trips + 1, seg, 0)

    pltpu.sync_copy(degacc.at[pl.ds(0, RS)], deg_hbm.at[pl.ds(w * RS, RS)])
    pltpu.sync_copy(sumacc.at[pl.ds(0, RS * 16)],
                    sumattr_hbm.at[pl.ds(w * RS * 16, RS * 16)])


@functools.partial(
    pl.kernel, mesh=_mesh, compiler_params=_CP,
    out_type=[
        jax.ShapeDtypeStruct((NP * 256,), jnp.float32),
        jax.ShapeDtypeStruct((NP * 16,), jnp.float32),
    ],
    scratch_types=[
        pltpu.VMEM((RPAD * 256,), jnp.float32),
        pltpu.VMEM((RPAD * 16,), jnp.float32),
        pltpu.VMEM((32, 256), jnp.float32),
        pltpu.VMEM((32, 256), jnp.float32),
        pltpu.VMEM((32, 256), jnp.float32),
        pltpu.VMEM((2 * SCH,), jnp.int32),
        pltpu.VMEM((2 * SCH,), jnp.int32),
        pltpu.VMEM((2 * SCH,), jnp.int32),
        pltpu.VMEM((256,), jnp.float32),
        pltpu.VMEM((16,), jnp.int32),
        pltpu.SemaphoreType.DMA,
        pltpu.SemaphoreType.DMA,
        pltpu.SemaphoreType.DMA,
    ],
)
def _sc_layer1(eidL, srcL, dlocL, counts, xl1_hbm, xr1_hbm, ea1_hbm, att_hbm,
               outsum_hbm, asum_hbm,
               acc, asumacc, xlb, xrb, eab, eidS, srcS, dlocS, attv,
               cntb, s1, s2, s3):
    w = _wid()
    lo = w * RS
    pltpu.sync_copy(att_hbm, attv)
    pltpu.sync_copy(counts.at[pl.ds(w * 16, 16)], cntb)
    cnt = cntb[pl.ds(0, 16)][0]

    def z1(i, _):
        acc[pl.ds(i * 16, 16)] = jnp.zeros((16,), jnp.float32)
        return 0
    lax.fori_loop(0, RPAD * 256 // 16, z1, 0)

    def z2(i, _):
        asumacc[pl.ds(i * 16, 16)] = jnp.zeros((16,), jnp.float32)
        return 0
    lax.fori_loop(0, RPAD, z2, 0)

    nchunks = ((cnt + SCH - 1) // SCH) * (SCH // 16)

    def body(k, _):
        @pl.when(k < nchunks)
        def _issue():
            ksup = k >> 5

            @pl.when((k & 31) == 0)
            def _ld():
                soff = w * CAP + ksup * SCH
                sl = (ksup & 1) * SCH
                pltpu.sync_copy(eidL.at[pl.ds(soff, SCH)], eidS.at[pl.ds(sl, SCH)])
                pltpu.sync_copy(srcL.at[pl.ds(soff, SCH)], srcS.at[pl.ds(sl, SCH)])
                pltpu.sync_copy(dlocL.at[pl.ds(soff, SCH)], dlocS.at[pl.ds(sl, SCH)])

            sl = (ksup & 1) * SCH + (k & 31) * 16
            srcv = srcS[pl.ds(sl, 16)]
            dstv = dlocS[pl.ds(sl, 16)] + lo
            eidv = eidS[pl.ds(sl, 16)]
            b = (k & 1) * 16
            pltpu.async_copy(xl1_hbm.at[srcv], xlb.at[pl.ds(b, 16)], s1)
            pltpu.async_copy(xr1_hbm.at[dstv], xrb.at[pl.ds(b, 16)], s2)
            pltpu.async_copy(ea1_hbm.at[eidv], eab.at[pl.ds(b, 16)], s3)

        @pl.when(k > 0)
        def _comp():
            kp = k - 1
            b = (kp & 1) * 16
            pltpu.make_async_copy(xl1_hbm.at[pl.ds(0, 16)],
                                  xlb.at[pl.ds(b, 16)], s1).wait()
            pltpu.make_async_copy(xr1_hbm.at[pl.ds(0, 16)],
                                  xrb.at[pl.ds(b, 16)], s2).wait()
            pltpu.make_async_copy(ea1_hbm.at[pl.ds(0, 16)],
                                  eab.at[pl.ds(b, 16)], s3).wait()
            slp = ((kp >> 5) & 1) * SCH + (kp & 31) * 16
            dlocv = dlocS[pl.ds(slp, 16)]
            lanes = _iota()
            for e in range(16):
                row = b + e
                dloc = dlocv[e]
                w4m = jnp.zeros((16,), jnp.float32)
                for h in range(4):
                    hacc = jnp.zeros((16,), jnp.float32)
                    xls = []
                    for q in range(4):
                        off = h * 64 + q * 16
                        xlv = xlb[row, pl.ds(off, 16)]
                        xls.append(xlv)
                        z = xlv + xrb[row, pl.ds(off, 16)] + eab[row, pl.ds(off, 16)]
                        lr = jnp.maximum(z, 0.2 * z)
                        hacc = hacc + lr * attv[pl.ds(off, 16)]
                    s = lax.reduce_sum(hacc, axes=(0,))
                    whb = jnp.exp(jnp.full((16,), s))
                    w4m = w4m + jnp.where(lanes == h, whb, 0.0)
                    for q in range(4):
                        off = h * 64 + q * 16
                        plsc.addupdate(acc.at[pl.ds(dloc * 256 + off, 16)],
                                       whb * xls[q])
                plsc.addupdate(asumacc.at[pl.ds(dloc * 16, 16)], w4m)
        return 0
    lax.fori_loop(0, nchunks + 1, body, 0)

    pltpu.sync_copy(acc.at[pl.ds(0, RS * 256)],
                    outsum_hbm.at[pl.ds(w * RS * 256, RS * 256)])
    pltpu.sync_copy(asumacc.at[pl.ds(0, RS * 16)],
                    asum_hbm.at[pl.ds(w * RS * 16, RS * 16)])


@functools.partial(
    pl.kernel, mesh=_mesh, compiler_params=_CP,
    out_type=[
        jax.ShapeDtypeStruct((NP * 64,), jnp.float32),
        jax.ShapeDtypeStruct((NP,), jnp.float32),
    ],
    scratch_types=[
        pltpu.VMEM((RPAD * 64,), jnp.float32),
        pltpu.VMEM((RPAD,), jnp.float32),
        pltpu.VMEM((RS, 128), jnp.float32),
        pltpu.VMEM((32, 128), jnp.float32),
        pltpu.VMEM((32, 128), jnp.float32),
        pltpu.VMEM((2 * SCH,), jnp.int32),
        pltpu.VMEM((2 * SCH,), jnp.int32),
        pltpu.VMEM((2 * SCH,), jnp.int32),
        pltpu.VMEM((64,), jnp.float32),
        pltpu.VMEM((16,), jnp.int32),
        pltpu.SemaphoreType.DMA,
        pltpu.SemaphoreType.DMA,
    ],
)
def _sc_layer2(eidL, srcL, dlocL, counts, t2_hbm, ea2_hbm, att_hbm,
               outsum_hbm, asum_hbm,
               acc, asumacc, t2r, tsb, eab, eidS, srcS, dlocS, attv,
               cntb, s1, s3):
    w = _wid()
    lo = w * RS
    pltpu.sync_copy(att_hbm, attv)
    pltpu.sync_copy(counts.at[pl.ds(w * 16, 16)], cntb)
    cnt = cntb[pl.ds(0, 16)][0]
    pltpu.sync_copy(t2_hbm.at[pl.ds(lo, RS)], t2r)

    def z1(i, _):
        acc[pl.ds(i * 16, 16)] = jnp.zeros((16,), jnp.float32)
        return 0
    lax.fori_loop(0, RPAD * 64 // 16, z1, 0)

    def z2(i, _):
        asumacc[pl.ds(i * 16, 16)] = jnp.zeros((16,), jnp.float32)
        return 0
    lax.fori_loop(0, RPAD // 8 // 2, z2, 0)

    nchunks = ((cnt + SCH - 1) // SCH) * (SCH // 16)

    def body(k, _):
        @pl.when(k < nchunks)
        def _issue():
            ksup = k >> 5

            @pl.when((k & 31) == 0)
            def _ld():
                soff = w * CAP + ksup * SCH
                sl = (ksup & 1) * SCH
                pltpu.sync_copy(eidL.at[pl.ds(soff, SCH)], eidS.at[pl.ds(sl, SCH)])
                pltpu.sync_copy(srcL.at[pl.ds(soff, SCH)], srcS.at[pl.ds(sl, SCH)])
                pltpu.sync_copy(dlocL.at[pl.ds(soff, SCH)], dlocS.at[pl.ds(sl, SCH)])

            sl = (ksup & 1) * SCH + (k & 31) * 16
            srcv = srcS[pl.ds(sl, 16)]
            eidv = eidS[pl.ds(sl, 16)]
            b = (k & 1) * 16
            pltpu.async_copy(t2_hbm.at[srcv], tsb.at[pl.ds(b, 16)], s1)
            pltpu.async_copy(ea2_hbm.at[eidv], eab.at[pl.ds(b, 16)], s3)

        @pl.when(k > 0)
        def _comp():
            kp = k - 1
            b = (kp & 1) * 16
            pltpu.make_async_copy(t2_hbm.at[pl.ds(0, 16)],
                                  tsb.at[pl.ds(b, 16)], s1).wait()
            pltpu.make_async_copy(ea2_hbm.at[pl.ds(0, 16)],
                                  eab.at[pl.ds(b, 16)], s3).wait()
            slp = ((kp >> 5) & 1) * SCH + (kp & 31) * 16
            dlocv = dlocS[pl.ds(slp, 16)]
            lanes = _iota()
            for e in range(16):
                row = b + e
                dloc = dlocv[e]
                hacc = jnp.zeros((16,), jnp.float32)
                xls = []
                for q in range(4):
                    off = q * 16
                    xlv = tsb[row, pl.ds(off, 16)]
                    xls.append(xlv)
                    z = xlv + t2r[dloc, pl.ds(off + 64, 16)] + eab[row, pl.ds(off, 16)]
                    lr = jnp.maximum(z, 0.2 * z)
                    hacc = hacc + lr * attv[pl.ds(off, 16)]
                s = lax.reduce_sum(hacc, axes=(0,))
                whb = jnp.exp(jnp.full((16,), s))
                plsc.addupdate_scatter(asumacc, [jnp.full((16,), dloc)], whb,
                                       mask=lanes == 0)
                for q in range(4):
                    off = q * 16
                    plsc.addupdate(acc.at[pl.ds(dloc * 64 + off, 16)],
                                   whb * xls[q])
        return 0
    lax.fori_loop(0, nchunks + 1, body, 0)

    pltpu.sync_copy(acc.at[pl.ds(0, RS * 64)],
                    outsum_hbm.at[pl.ds(w * RS * 64, RS * 64)])
    pltpu.sync_copy(asumacc.at[pl.ds(0, RS)], asum_hbm.at[pl.ds(w * RS, RS)])


def _tc_nodes(xp, Wl1, bl1, Wr1, br1):
    def body(x_ref, wl_ref, bl_ref, wr_ref, br_ref, ol_ref, or_ref):
        xb = x_ref[...]
        ol_ref[...] = jnp.dot(xb, wl_ref[...],
                              preferred_element_type=jnp.float32, precision=lax.Precision.HIGHEST) + bl_ref[...]
        or_ref[...] = jnp.dot(xb, wr_ref[...],
                              preferred_element_type=jnp.float32, precision=lax.Precision.HIGHEST) + br_ref[...]
    full = lambda s: pl.BlockSpec(s, lambda i: (0, 0))
    return pl.pallas_call(
        body,
        grid=(NP // 1024,),
        in_specs=[pl.BlockSpec((1024, 64), lambda i: (i, 0)),
                  full((64, 256)), full((1, 256)), full((64, 256)), full((1, 256))],
        out_specs=[pl.BlockSpec((1024, 256), lambda i: (i, 0)),
                   pl.BlockSpec((1024, 256), lambda i: (i, 0))],
        out_shape=[jax.ShapeDtypeStruct((NP, 256), jnp.float32),
                   jax.ShapeDtypeStruct((NP, 256), jnp.float32)],
    )(xp, Wl1, bl1.reshape(1, 256), Wr1, br1.reshape(1, 256))


def _tc_edges(eattr, We1, We2p):
    def body(e_ref, w1_ref, w2_ref, o1_ref, o2_ref):
        eb = e_ref[...]
        o1_ref[...] = jnp.dot(eb, w1_ref[...], preferred_element_type=jnp.float32, precision=lax.Precision.HIGHEST)
        o2_ref[...] = jnp.dot(eb, w2_ref[...], preferred_element_type=jnp.float32, precision=lax.Precision.HIGHEST)
    full = lambda s: pl.BlockSpec(s, lambda i: (0, 0))
    return pl.pallas_call(
        body,
        grid=(E // 8000,),
        in_specs=[pl.BlockSpec((8000, 16), lambda i: (i, 0)),
                  full((16, 256)), full((16, 128))],
        out_specs=[pl.BlockSpec((8000, 256), lambda i: (i, 0)),
                   pl.BlockSpec((8000, 128), lambda i: (i, 0))],
        out_shape=[jax.ShapeDtypeStruct((E, 256), jnp.float32),
                   jax.ShapeDtypeStruct((E, 128), jnp.float32)],
    )(eattr, We1, We2p)


def _tc_mid(outsum1, asum1, xl1, xr1, sumattr, deg, We1, We2, S, ST, b1,
            att1bc, att2col, Wl2, bl2, Wr2, br2):
    def body(os_ref, as_ref, xl_ref, xr_ref, sa_ref, dg_ref, we1_ref, we2_ref,
             s_ref, st_ref, b1_ref, a1_ref, a2_ref, wl2_ref, bl2_ref, wr2_ref,
             br2_ref, t2_ref, w2_ref):
        f32 = jnp.float32
        ma = sa_ref[...] / jnp.maximum(dg_ref[...], 1.0)
        la1 = jnp.dot(ma, we1_ref[...], preferred_element_type=f32, precision=lax.Precision.HIGHEST)
        xl = xl_ref[...]
        z = xl + xr_ref[...] + la1
        lr = jnp.maximum(z, 0.2 * z)
        logits = jnp.dot(lr * a1_ref[...], s_ref[...], preferred_element_type=f32, precision=lax.Precision.HIGHEST)
        wl1 = jnp.exp(logits)
        wl1b = jnp.dot(wl1, st_ref[...], preferred_element_type=f32, precision=lax.Precision.HIGHEST)
        num = os_ref[...] + wl1b * xl
        den = jnp.dot(as_ref[...][:, :4] + wl1, st_ref[...],
                      preferred_element_type=f32, precision=lax.Precision.HIGHEST)
        h1 = jnp.maximum(num / den + b1_ref[...], 0.0)
        xl2 = jnp.dot(h1, wl2_ref[...], preferred_element_type=f32, precision=lax.Precision.HIGHEST) + bl2_ref[...]
        xr2 = jnp.dot(h1, wr2_ref[...], preferred_element_type=f32, precision=lax.Precision.HIGHEST) + br2_ref[...]
        t2_ref[...] = jnp.concatenate([xl2, xr2], axis=1)
        la2 = jnp.dot(ma, we2_ref[...], preferred_element_type=f32, precision=lax.Precision.HIGHEST)
        z2 = xl2 + xr2 + la2
        lr2 = jnp.maximum(z2, 0.2 * z2)
        w2_ref[...] = jnp.exp(jnp.dot(lr2, a2_ref[...], preferred_element_type=f32, precision=lax.Precision.HIGHEST))
    full = lambda s: pl.BlockSpec(s, lambda i: (0, 0))
    blk = lambda s: pl.BlockSpec(s, lambda i: (i, 0))
    return pl.pallas_call(
        body,
        grid=(NP // 1024,),
        in_specs=[blk((1024, 256)), blk((1024, 16)), blk((1024, 256)),
                  blk((1024, 256)), blk((1024, 16)), blk((1024, 1)),
                  full((16, 256)), full((16, 64)), full((256, 4)),
                  full((4, 256)), full((1, 256)), full((1, 256)),
                  full((64, 1)), full((256, 64)), full((1, 64)),
                  full((256, 64)), full((1, 64))],
        out_specs=[blk((1024, 128)), blk((1024, 1))],
        out_shape=[jax.ShapeDtypeStruct((NP, 128), jnp.float32),
                   jax.ShapeDtypeStruct((NP, 1), jnp.float32)],
    )(outsum1, asum1, xl1, xr1, sumattr, deg, We1, We2, S, ST, b1, att1bc,
      att2col, Wl2, bl2, Wr2, br2)


def _tc_head(outsum2, asum2, wl2, T2, b2, Wh1, bh1, Wh2, bh2):
    def body(os_ref, as_ref, w2_ref, t2_ref, b2_ref, wh1_ref, bh1_ref,
             wh2_ref, bh2_ref, y_ref):
        f32 = jnp.float32
        xl2 = t2_ref[...][:, :64]
        w2 = w2_ref[...]
        out2 = (os_ref[...] + w2 * xl2) / (as_ref[...] + w2) + b2_ref[...]
        hh = jnp.maximum(jnp.dot(out2, wh1_ref[...], preferred_element_type=f32, precision=lax.Precision.HIGHEST)
                         + bh1_ref[...], 0.0)
        y_ref[...] = jnp.dot(hh, wh2_ref[...],
                             preferred_element_type=f32, precision=lax.Precision.HIGHEST) + bh2_ref[...]
    full = lambda s: pl.BlockSpec(s, lambda i: (0, 0))
    blk = lambda s: pl.BlockSpec(s, lambda i: (i, 0))
    return pl.pallas_call(
        body,
        grid=(NP // 1024,),
        in_specs=[blk((1024, 64)), blk((1024, 1)), blk((1024, 1)),
                  blk((1024, 128)), full((1, 64)), full((64, 64)),
                  full((1, 64)), full((64, 2)), full((1, 2))],
        out_specs=blk((1024, 2)),
        out_shape=jax.ShapeDtypeStruct((NP, 2), jnp.float32),
    )(outsum2, asum2, wl2, T2, b2.reshape(1, 64), Wh1, bh1.reshape(1, 64),
      Wh2, bh2.reshape(1, 2))


def kernel(x, edge_index, edge_attr, Wl1, bl1, Wr1, br1, We1, att1, b1,
           Wl2, bl2, Wr2, br2, We2, att2, b2, Wh1, bh1, Wh2, bh2):
    src = edge_index[0]
    dst = edge_index[1]
    xp = jnp.pad(x, ((0, NP - N), (0, 0)))
    ea128 = edge_attr.reshape(E // 8, 128)
    We2p = jnp.pad(We2, ((0, 0), (0, 64)))

    eidL, srcL, dlocL, counts, deg, sumattr_f = _sc_route(src, dst, ea128)
    sumattr = sumattr_f.reshape(NP, 16)

    xl1, xr1 = _tc_nodes(xp, Wl1, bl1, Wr1, br1)
    ea1, ea2p = _tc_edges(edge_attr, We1, We2p)

    outsum1_f, asum1_f = _sc_layer1(eidL, srcL, dlocL, counts, xl1, xr1, ea1,
                                    att1.reshape(256))
    outsum1 = outsum1_f.reshape(NP, 256)
    asum1 = asum1_f.reshape(NP, 16)

    S = jnp.kron(jnp.eye(4, dtype=jnp.float32), jnp.ones((64, 1), jnp.float32))
    T2, wl2 = _tc_mid(outsum1, asum1, xl1, xr1, sumattr, deg.reshape(NP, 1),
                      We1, We2, S, S.T, b1.reshape(1, 256),
                      att1.reshape(1, 256), att2.reshape(64, 1),
                      Wl2, bl2.reshape(1, 64), Wr2, br2.reshape(1, 64))

    outsum2_f, asum2_f = _sc_layer2(eidL, srcL, dlocL, counts, T2, ea2p,
                                    att2.reshape(64))
    outsum2 = outsum2_f.reshape(NP, 64)
    asum2 = asum2_f.reshape(NP, 1)

    y = _tc_head(outsum2, asum2, wl2, T2, b2, Wh1, bh1, Wh2, bh2)
    return y[:N]
